# Initial kernel scaffold; baseline (speedup 1.0000x reference)
import jax
import jax.numpy as jnp
from jax.experimental import pallas as pl


def kernel(z, pos, batch, idx_i, idx_j, embedding, in2f_W, filt_W1, filt_b1, filt_W2, filt_b2, f2out_W1, f2out_b1, f2out_W2, f2out_b2, clf_W, clf_b):
    raise NotImplementedError("write your pallas kernel here")


# speedup vs baseline: 4.4722x; 4.4722x over previous
import functools

import jax
import jax.numpy as jnp
from jax import lax
from jax.experimental import pallas as pl
from jax.experimental.pallas import tpu as pltpu
from jax.experimental.pallas import tpu_sc as plsc

N = 100000
E = 3200000
H = 10
NRBF = 30
NMOL = 5000
NLAB = 11
CUTOFF = 5.0
MAXZ = 100
NINT = 3

HP = 16
NC, NS, L = 2, 16, 16
NW = NC * NS
NODE_PAD = 100352
NB = NODE_PAD // 2048
SPT = NODE_PAD // NS
E_PAD = 3276800
EPW = E_PAD // NW
EB = 1024
NBLK = EPW // EB
NG = EB // L
K = 2048
TROWS = 2056
SCALE = K / CUTOFF
MOL_PAD = 6144
MPT = MOL_PAD // NS

_mesh = plsc.VectorSubcoreMesh(core_axis_name="c", subcore_axis_name="s")
_sc_params = pltpu.CompilerParams(needs_layout_passes=False)


def _sp(x):
    return jax.nn.softplus(x) - jnp.log(2.0)


def _tables_body(w1_ref, b1_ref, w2_ref, b2_ref, out_ref):
    dgrid = lax.broadcasted_iota(jnp.int32, (TROWS, 1), 0).astype(jnp.float32) * (
        CUTOFF / K)
    offs = lax.broadcasted_iota(jnp.int32, (1, NRBF), 1).astype(jnp.float32) * (
        CUTOFF / (NRBF - 1))
    width = CUTOFF / (NRBF - 1)
    coeff = -0.5 / (width * width)
    fg = jnp.exp(coeff * (dgrid - offs) ** 2)
    rc = 0.5 * (jnp.cos(dgrid * (jnp.pi / CUTOFF)) + 1.0)
    rc = rc * (dgrid < CUTOFF).astype(jnp.float32)
    for t in range(NINT):
        pre = jnp.dot(fg, w1_ref[t], preferred_element_type=jnp.float32) + b1_ref[t]
        wf = jnp.dot(_sp(pre), w2_ref[t], preferred_element_type=jnp.float32) + b2_ref[t]
        out_ref[t] = wf * rc


def _build_tables(w1, b1, w2p, b2p):
    return pl.pallas_call(
        _tables_body,
        out_shape=jax.ShapeDtypeStruct((NINT, TROWS, HP), jnp.float32),
    )(w1, b1, w2p, b2p)


def _x0_body(z_ref, emb_ref, w_ref, x_ref, xf_ref):
    zv = z_ref[0, 0]
    oh = (zv[:, None] == lax.broadcasted_iota(jnp.int32, (2048, 128), 1))
    x0 = jnp.dot(oh.astype(jnp.float32), emb_ref[...],
                 preferred_element_type=jnp.float32)
    xf = jnp.dot(x0, w_ref[...], preferred_element_type=jnp.float32)
    x_ref[...] = x0
    for f in range(H):
        xf_ref[f, 0, 0] = xf[:, f]


def _x0_xf0(z3d, embp, in2f0):
    return pl.pallas_call(
        _x0_body,
        grid=(NB,),
        in_specs=[
            pl.BlockSpec((1, 1, 2048), lambda i: (i, 0, 0)),
            pl.BlockSpec((128, HP), lambda i: (0, 0)),
            pl.BlockSpec((HP, HP), lambda i: (0, 0)),
        ],
        out_specs=[
            pl.BlockSpec((2048, HP), lambda i: (i, 0)),
            pl.BlockSpec((H, 1, 1, 2048), lambda i: (0, i, 0, 0)),
        ],
        out_shape=[
            jax.ShapeDtypeStruct((NODE_PAD, HP), jnp.float32),
            jax.ShapeDtypeStruct((H, NB, 1, 2048), jnp.float32),
        ],
    )(z3d, embp, in2f0)


def _prep_body(px_hbm, py_hbm, pz_hbm, ii_hbm, ij_hbm, u_hbm,
               ii_v, ij_v, pix, piy, piz, pjx, pjy, pjz, u_v, sem):
    cid = lax.axis_index("c")
    sid = lax.axis_index("s")
    w = sid * NC + cid
    iota = lax.iota(jnp.int32, L)
    kf32 = jnp.float32(K)

    def block(blk, _):
        ebase = w * EPW + blk * EB
        pltpu.sync_copy(ii_hbm.at[pl.ds(ebase, EB)], ii_v)
        pltpu.sync_copy(ij_hbm.at[pl.ds(ebase, EB)], ij_v)
        descs = [
            pltpu.async_copy(px_hbm.at[ii_v], pix, sem),
            pltpu.async_copy(py_hbm.at[ii_v], piy, sem),
            pltpu.async_copy(pz_hbm.at[ii_v], piz, sem),
            pltpu.async_copy(px_hbm.at[ij_v], pjx, sem),
            pltpu.async_copy(py_hbm.at[ij_v], pjy, sem),
            pltpu.async_copy(pz_hbm.at[ij_v], pjz, sem),
        ]
        for d in descs:
            d.wait()

        def group(g, _):
            sl = pl.ds(g * L, L)
            dx = pjx[sl] - pix[sl]
            dy = pjy[sl] - piy[sl]
            dz = pjz[sl] - piz[sl]
            s = dx * dx + dy * dy + dz * dz + 1e-12
            bi = plsc.bitcast(s, jnp.int32)
            y = plsc.bitcast((bi >> 1) + 0x1FBD1DF5, jnp.float32)
            y = 0.5 * (y + s / y)
            y = 0.5 * (y + s / y)
            y = 0.5 * (y + s / y)
            u = jnp.minimum(y * SCALE, kf32)
            gid = w * EPW + blk * EB + g * L + iota
            u = jnp.where(gid < E, u, kf32)
            u_v[sl] = u
            return _

        lax.fori_loop(0, NG, group, None)
        pltpu.sync_copy(u_v, u_hbm.at[pl.ds(ebase, EB)])
        return _

    lax.fori_loop(0, NBLK, block, None)


@functools.partial(
    pl.kernel,
    out_type=jax.ShapeDtypeStruct((E_PAD,), jnp.float32),
    mesh=_mesh,
    scratch_types=[
        pltpu.VMEM((EB,), jnp.int32),
        pltpu.VMEM((EB,), jnp.int32),
        pltpu.VMEM((EB,), jnp.float32),
        pltpu.VMEM((EB,), jnp.float32),
        pltpu.VMEM((EB,), jnp.float32),
        pltpu.VMEM((EB,), jnp.float32),
        pltpu.VMEM((EB,), jnp.float32),
        pltpu.VMEM((EB,), jnp.float32),
        pltpu.VMEM((EB,), jnp.float32),
        pltpu.SemaphoreType.DMA,
    ],
    compiler_params=_sc_params,
)
def _prep(px_hbm, py_hbm, pz_hbm, ii_hbm, ij_hbm, u_hbm,
          ii_v, ij_v, pix, piy, piz, pjx, pjy, pjz, u_v, sem):
    _prep_body(px_hbm, py_hbm, pz_hbm, ii_hbm, ij_hbm, u_hbm,
               ii_v, ij_v, pix, piy, piz, pjx, pjy, pjz, u_v, sem)


def _edge_body(u_hbm, ij_hbm, ii_hbm, xf_hbms, t_hbm, z_hbm, agg_hbm,
               t_v, ij_v, ii_v, u_v, ti_v, fr_v, xfj_v, msg_v, agg_shs,
               semg, sems):
    cid = lax.axis_index("c")
    sid = lax.axis_index("s")
    w = sid * NC + cid

    pltpu.sync_copy(t_hbm, t_v)
    for f in range(H):
        pltpu.sync_copy(z_hbm, agg_shs[f].at[pl.ds(sid * SPT, SPT)])
    plsc.subcore_barrier()

    def block(blk, _):
        ebase = w * EPW + blk * EB
        pltpu.sync_copy(ij_hbm.at[pl.ds(ebase, EB)], ij_v)
        pltpu.sync_copy(ii_hbm.at[pl.ds(ebase, EB)], ii_v)
        pltpu.sync_copy(u_hbm.at[pl.ds(ebase, EB)], u_v)

        def pre(g, _):
            sl = pl.ds(g * L, L)
            uv = u_v[sl]
            ki = jnp.minimum(uv.astype(jnp.int32), K - 1)
            fr_v[sl] = uv - ki.astype(jnp.float32)
            ti_v[sl] = ki * HP
            return _

        lax.fori_loop(0, NG, pre, None)

        gat = [None, None]
        sca = [None, None]
        gat[0] = pltpu.async_copy(xf_hbms[0].at[ij_v], xfj_v[0], semg[0])
        for f in range(H):
            b = f % 2
            gat[b].wait()
            if f + 1 < H:
                gat[1 - b] = pltpu.async_copy(
                    xf_hbms[f + 1].at[ij_v], xfj_v[1 - b], semg[1 - b])
            if sca[b] is not None:
                sca[b].wait()

            def group(g, _):
                sl = pl.ds(g * L, L)
                tb = ti_v[sl] + f
                a = plsc.load_gather(t_v, [tb])
                bb = plsc.load_gather(t_v, [tb + HP])
                wf = a + fr_v[sl] * (bb - a)
                msg_v[b][sl] = xfj_v[b][sl] * wf
                return _

            lax.fori_loop(0, NG, group, None)
            sca[b] = pltpu.async_copy(
                msg_v[b], agg_shs[f].at[ii_v], sems[b], add=True)
        for d in sca:
            if d is not None:
                d.wait()
        return _

    lax.fori_loop(0, NBLK, block, None)
    plsc.subcore_barrier()
    for f in range(H):
        pltpu.sync_copy(
            agg_shs[f].at[pl.ds(sid * SPT, SPT)],
            agg_hbm.at[pl.ds((cid * H + f) * NODE_PAD + sid * SPT, SPT)])


@functools.partial(
    pl.kernel,
    out_type=jax.ShapeDtypeStruct((NC * H * NODE_PAD,), jnp.float32),
    mesh=_mesh,
    scratch_types=[
        pltpu.VMEM((TROWS * HP,), jnp.float32),
        pltpu.VMEM((EB,), jnp.int32),
        pltpu.VMEM((EB,), jnp.int32),
        pltpu.VMEM((EB,), jnp.float32),
        pltpu.VMEM((EB,), jnp.int32),
        pltpu.VMEM((EB,), jnp.float32),
        pltpu.VMEM((EB,), jnp.float32),
        pltpu.VMEM((EB,), jnp.float32),
        pltpu.VMEM((EB,), jnp.float32),
        pltpu.VMEM((EB,), jnp.float32),
    ] + [pltpu.VMEM_SHARED((NODE_PAD,), jnp.float32)] * H + [
        pltpu.SemaphoreType.DMA,
        pltpu.SemaphoreType.DMA,
        pltpu.SemaphoreType.DMA,
        pltpu.SemaphoreType.DMA,
    ],
    compiler_params=_sc_params,
)
def _edge_pass(u_hbm, ij_hbm, ii_hbm, xf0, xf1, xf2, xf3, xf4, xf5, xf6,
               xf7, xf8, xf9, t_hbm, z_hbm, agg_hbm,
               t_v, ij_v, ii_v, u_v, ti_v, fr_v, xfj0, xfj1, msg0, msg1,
               a0, a1, a2, a3, a4, a5, a6, a7, a8, a9,
               semg0, semg1, sems0, sems1):
    _edge_body(u_hbm, ij_hbm, ii_hbm,
               [xf0, xf1, xf2, xf3, xf4, xf5, xf6, xf7, xf8, xf9],
               t_hbm, z_hbm, agg_hbm,
               t_v, ij_v, ii_v, u_v, ti_v, fr_v, [xfj0, xfj1], [msg0, msg1],
               [a0, a1, a2, a3, a4, a5, a6, a7, a8, a9],
               [semg0, semg1], [sems0, sems1])


def _node_body(nxt_w_ref, w1_ref, b1_ref, w2_ref, b2_ref, agg_ref, x_ref,
               xn_ref, xf_ref):
    cols = [agg_ref[f, 0, 0] + agg_ref[H + f, 0, 0] for f in range(H)]
    agg10 = jnp.stack(cols, axis=1)
    agg = jnp.concatenate(
        [agg10, jnp.zeros((2048, HP - H), jnp.float32)], axis=1)
    pre = jnp.dot(agg, w1_ref[...], preferred_element_type=jnp.float32) + b1_ref[...]
    v = jnp.dot(_sp(pre), w2_ref[...], preferred_element_type=jnp.float32) + b2_ref[...]
    xn = x_ref[...] + v
    xn_ref[...] = xn
    xf = jnp.dot(xn, nxt_w_ref[...], preferred_element_type=jnp.float32)
    for f in range(H):
        xf_ref[f, 0, 0] = xf[:, f]


def _node_pass(agg4, x, w1p, b1p, w2p, b2p, nxt_w):
    wspec = pl.BlockSpec((HP, HP), lambda i: (0, 0))
    bspec = pl.BlockSpec((HP,), lambda i: (0,))
    nspec = pl.BlockSpec((2048, HP), lambda i: (i, 0))
    return pl.pallas_call(
        _node_body,
        grid=(NB,),
        in_specs=[wspec, wspec, bspec, wspec, bspec,
                  pl.BlockSpec((2 * H, 1, 1, 2048), lambda i: (0, i, 0, 0)),
                  nspec],
        out_specs=[nspec,
                   pl.BlockSpec((H, 1, 1, 2048), lambda i: (0, i, 0, 0))],
        out_shape=[jax.ShapeDtypeStruct((NODE_PAD, HP), jnp.float32),
                   jax.ShapeDtypeStruct((H, NB, 1, 2048), jnp.float32)],
    )(nxt_w, w1p, b1p, w2p, b2p, agg4, x)


def _final_body(x_ref, xp_ref):
    i = pl.program_id(0)
    xn = x_ref[...]
    rowid = i * 2048 + lax.broadcasted_iota(jnp.int32, (2048, HP), 0)
    xn = jnp.where(rowid < N, xn, 0.0)
    for f in range(H):
        xp_ref[f, 0, 0] = xn[:, f]


def _final_planes(x):
    return pl.pallas_call(
        _final_body,
        grid=(NB,),
        in_specs=[pl.BlockSpec((2048, HP), lambda i: (i, 0))],
        out_specs=pl.BlockSpec((H, 1, 1, 2048), lambda i: (0, i, 0, 0)),
        out_shape=jax.ShapeDtypeStruct((H, NB, 1, 2048), jnp.float32),
    )(x)


def _readout_body(xp_hbm, b_hbm, z_hbm, one_hbm, mol_hbm,
                  b_v, x_v, one_v, mol_shs, sem):
    cid = lax.axis_index("c")
    sid = lax.axis_index("s")
    w = sid * NC + cid
    nsb = NODE_PAD // EB
    cnt = (nsb - 1 - w) // NW + 1
    for f in range(H + 1):
        pltpu.sync_copy(z_hbm, mol_shs[f].at[pl.ds(sid * MPT, MPT)])
    pltpu.sync_copy(one_hbm, one_v)
    plsc.subcore_barrier()

    def block(q, _):
        sb = w + q * NW
        pltpu.sync_copy(b_hbm.at[pl.ds(sb * EB, EB)], b_v)
        for f in range(H):
            pltpu.sync_copy(xp_hbm.at[pl.ds(f * NODE_PAD + sb * EB, EB)], x_v)
            pltpu.sync_copy(x_v, mol_shs[f].at[b_v], add=True)
        pltpu.sync_copy(one_v, mol_shs[H].at[b_v], add=True)
        return _

    lax.fori_loop(0, cnt, block, None)
    plsc.subcore_barrier()
    for f in range(H + 1):
        pltpu.sync_copy(
            mol_shs[f].at[pl.ds(sid * MPT, MPT)],
            mol_hbm.at[pl.ds((cid * (H + 1) + f) * MOL_PAD + sid * MPT, MPT)])


@functools.partial(
    pl.kernel,
    out_type=jax.ShapeDtypeStruct((NC * (H + 1) * MOL_PAD,), jnp.float32),
    mesh=_mesh,
    scratch_types=[
        pltpu.VMEM((EB,), jnp.int32),
        pltpu.VMEM((EB,), jnp.float32),
        pltpu.VMEM((EB,), jnp.float32),
    ] + [pltpu.VMEM_SHARED((MOL_PAD,), jnp.float32)] * (H + 1) + [
        pltpu.SemaphoreType.DMA,
    ],
    compiler_params=_sc_params,
)
def _readout(xp_hbm, b_hbm, z_hbm, one_hbm, mol_hbm, b_v, x_v, one_v,
             m0, m1, m2, m3, m4, m5, m6, m7, m8, m9, m10, sem):
    _readout_body(xp_hbm, b_hbm, z_hbm, one_hbm, mol_hbm, b_v, x_v, one_v,
                  [m0, m1, m2, m3, m4, m5, m6, m7, m8, m9, m10], sem)


def _head_body(molp_ref, w_ref, b_ref, out_ref):
    cols = [molp_ref[f] + molp_ref[H + 1 + f] for f in range(H + 1)]
    counts = jnp.maximum(cols[H], 1.0)[:, None]
    h10 = jnp.stack(cols[:H], axis=1) / counts
    out_ref[...] = jnp.dot(h10, w_ref[...],
                           preferred_element_type=jnp.float32) + b_ref[...]


def _head(molp, clf_w, clf_b):
    return pl.pallas_call(
        _head_body,
        out_shape=jax.ShapeDtypeStruct((MOL_PAD, NLAB), jnp.float32),
    )(molp, clf_w, clf_b)


def kernel(z, pos, batch, idx_i, idx_j, embedding, in2f_W, filt_W1, filt_b1,
           filt_W2, filt_b2, f2out_W1, f2out_b1, f2out_W2, f2out_b2, clf_W, clf_b):
    f32 = jnp.float32
    z3d = jnp.pad(z.astype(jnp.int32), (0, NODE_PAD - N)).reshape(-1, 1, 2048)
    posf = pos.astype(f32)
    px = jnp.pad(posf[:, 0], (0, NODE_PAD - N))
    py = jnp.pad(posf[:, 1], (0, NODE_PAD - N))
    pz = jnp.pad(posf[:, 2], (0, NODE_PAD - N))
    ii1 = jnp.pad(idx_i.astype(jnp.int32), (0, E_PAD - E))
    ij1 = jnp.pad(idx_j.astype(jnp.int32), (0, E_PAD - E))
    b1 = jnp.pad(batch.astype(jnp.int32), (0, NODE_PAD - N),
                 constant_values=NMOL)
    embp = jnp.pad(embedding.astype(f32), ((0, 128 - MAXZ), (0, HP - H)))
    in2fp = jnp.pad(in2f_W.astype(f32), ((0, 0), (0, HP - H), (0, HP - H)))
    fw2p = jnp.pad(filt_W2.astype(f32), ((0, 0), (0, 0), (0, HP - H)))
    fb2p = jnp.pad(filt_b2.astype(f32), ((0, 0), (0, HP - H)))
    ow1p = jnp.pad(f2out_W1.astype(f32), ((0, 0), (0, HP - H), (0, HP - H)))
    ob1p = jnp.pad(f2out_b1.astype(f32), ((0, 0), (0, HP - H)))
    ow2p = jnp.pad(f2out_W2.astype(f32), ((0, 0), (0, HP - H), (0, HP - H)))
    ob2p = jnp.pad(f2out_b2.astype(f32), ((0, 0), (0, HP - H)))
    zplane = jnp.zeros((SPT,), f32)
    zmol = jnp.zeros((MPT,), f32)
    ones = jnp.ones((EB,), f32)

    tables = _build_tables(filt_W1.astype(f32), filt_b1.astype(f32), fw2p, fb2p)
    tables2 = tables.reshape(NINT, TROWS * HP)
    x, xf4 = _x0_xf0(z3d, embp, in2fp[0])
    u = _prep(px, py, pz, ii1, ij1)
    nxt = jnp.roll(in2fp, -1, axis=0)

    niter = (jnp.float32(NINT) + jnp.min(posf) * 0.0).astype(jnp.int32)

    def step(t, carry):
        xc, xf4c = carry
        xfp = xf4c.reshape(H, NODE_PAD)
        agg = _edge_pass(u, ij1, ii1, *[xfp[f] for f in range(H)],
                         tables2[t], zplane)
        agg4 = agg.reshape(2 * H, NB, 1, 2048)
        xc, xf4c = _node_pass(agg4, xc, ow1p[t], ob1p[t], ow2p[t], ob2p[t],
                              nxt[t])
        return (xc, xf4c)

    x, _ = lax.fori_loop(0, niter, step, (x, xf4))
    xp = _final_planes(x).reshape(H * NODE_PAD)

    molp = _readout(xp, b1, zmol, ones)
    logits = _head(molp.reshape(NC * (H + 1), MOL_PAD), clf_W.astype(f32),
                   clf_b.astype(f32))
    return logits[:NMOL]


# speedup vs baseline: 6.8713x; 1.5364x over previous
import functools

import jax
import jax.numpy as jnp
from jax import lax
from jax.experimental import pallas as pl
from jax.experimental.pallas import tpu as pltpu
from jax.experimental.pallas import tpu_sc as plsc

N = 100000
E = 3200000
H = 10
NRBF = 30
NMOL = 5000
NLAB = 11
CUTOFF = 5.0
MAXZ = 100
NINT = 3

HP = 16
NC, NS, L = 2, 16, 16
NW = NC * NS
NODE_PAD = 100352
NB = NODE_PAD // 2048
SPT = NODE_PAD // NS
E_PAD = 3276800
EPW = E_PAD // NW
EB = 1024
NBLK = EPW // EB
NG = EB // L
K = 2048
TROWS = 2056
SCALE = K / CUTOFF
MOL_PAD = 6144
MPT = MOL_PAD // NS

_mesh = plsc.VectorSubcoreMesh(core_axis_name="c", subcore_axis_name="s")
_sc_params = pltpu.CompilerParams(needs_layout_passes=False)


def _sp(x):
    return jax.nn.softplus(x) - jnp.log(2.0)


def _tables_body(w1_ref, b1_ref, w2_ref, b2_ref, out_ref):
    dgrid = lax.broadcasted_iota(jnp.int32, (TROWS, 1), 0).astype(jnp.float32) * (
        CUTOFF / K)
    offs = lax.broadcasted_iota(jnp.int32, (1, NRBF), 1).astype(jnp.float32) * (
        CUTOFF / (NRBF - 1))
    width = CUTOFF / (NRBF - 1)
    coeff = -0.5 / (width * width)
    fg = jnp.exp(coeff * (dgrid - offs) ** 2)
    rc = 0.5 * (jnp.cos(dgrid * (jnp.pi / CUTOFF)) + 1.0)
    rc = rc * (dgrid < CUTOFF).astype(jnp.float32)
    for t in range(NINT):
        pre = jnp.dot(fg, w1_ref[t], preferred_element_type=jnp.float32) + b1_ref[t]
        wf = jnp.dot(_sp(pre), w2_ref[t], preferred_element_type=jnp.float32) + b2_ref[t]
        out_ref[t] = wf * rc


def _build_tables(w1, b1, w2p, b2p):
    return pl.pallas_call(
        _tables_body,
        out_shape=jax.ShapeDtypeStruct((NINT, TROWS, HP), jnp.float32),
    )(w1, b1, w2p, b2p)


def _pack_pairs(xf, xf_ref):
    u16 = lax.bitcast_convert_type(
        xf.astype(jnp.bfloat16), jnp.uint16).astype(jnp.uint32)
    for p in range(H // 2):
        pk = u16[:, 2 * p] | (u16[:, 2 * p + 1] << 16)
        xf_ref[p, 0, 0] = lax.bitcast_convert_type(pk, jnp.float32)


def _x0_body(z_ref, emb_ref, w_ref, x_ref, xf_ref):
    zv = z_ref[0, 0]
    oh = (zv[:, None] == lax.broadcasted_iota(jnp.int32, (2048, 128), 1))
    x0 = jnp.dot(oh.astype(jnp.float32), emb_ref[...],
                 preferred_element_type=jnp.float32)
    xf = jnp.dot(x0, w_ref[...], preferred_element_type=jnp.float32)
    x_ref[...] = x0
    _pack_pairs(xf, xf_ref)


def _x0_xf0(z3d, embp, in2f0):
    return pl.pallas_call(
        _x0_body,
        grid=(NB,),
        in_specs=[
            pl.BlockSpec((1, 1, 2048), lambda i: (i, 0, 0)),
            pl.BlockSpec((128, HP), lambda i: (0, 0)),
            pl.BlockSpec((HP, HP), lambda i: (0, 0)),
        ],
        out_specs=[
            pl.BlockSpec((2048, HP), lambda i: (i, 0)),
            pl.BlockSpec((H // 2, 1, 1, 2048), lambda i: (0, i, 0, 0)),
        ],
        out_shape=[
            jax.ShapeDtypeStruct((NODE_PAD, HP), jnp.float32),
            jax.ShapeDtypeStruct((H // 2, NB, 1, 2048), jnp.float32),
        ],
    )(z3d, embp, in2f0)


def _prep_body(px_hbm, py_hbm, pz_hbm, ii_hbm, ij_hbm, u_hbm,
               ii_v, ij_v, pix, piy, piz, pjx, pjy, pjz, u_v, sem):
    cid = lax.axis_index("c")
    sid = lax.axis_index("s")
    w = sid * NC + cid
    iota = lax.iota(jnp.int32, L)
    kf32 = jnp.float32(K)

    def block(blk, _):
        ebase = w * EPW + blk * EB
        pltpu.sync_copy(ii_hbm.at[pl.ds(ebase, EB)], ii_v)
        pltpu.sync_copy(ij_hbm.at[pl.ds(ebase, EB)], ij_v)
        descs = [
            pltpu.async_copy(px_hbm.at[ii_v], pix, sem),
            pltpu.async_copy(py_hbm.at[ii_v], piy, sem),
            pltpu.async_copy(pz_hbm.at[ii_v], piz, sem),
            pltpu.async_copy(px_hbm.at[ij_v], pjx, sem),
            pltpu.async_copy(py_hbm.at[ij_v], pjy, sem),
            pltpu.async_copy(pz_hbm.at[ij_v], pjz, sem),
        ]
        for d in descs:
            d.wait()

        def group(g, _):
            sl = pl.ds(g * L, L)
            dx = pjx[sl] - pix[sl]
            dy = pjy[sl] - piy[sl]
            dz = pjz[sl] - piz[sl]
            s = dx * dx + dy * dy + dz * dz + 1e-12
            bi = plsc.bitcast(s, jnp.int32)
            y = plsc.bitcast((bi >> 1) + 0x1FBD1DF5, jnp.float32)
            y = 0.5 * (y + s / y)
            y = 0.5 * (y + s / y)
            y = 0.5 * (y + s / y)
            u = jnp.minimum(y * SCALE, kf32)
            gid = w * EPW + blk * EB + g * L + iota
            u = jnp.where(gid < E, u, kf32)
            u_v[sl] = u
            return _

        lax.fori_loop(0, NG, group, None)
        pltpu.sync_copy(u_v, u_hbm.at[pl.ds(ebase, EB)])
        return _

    lax.fori_loop(0, NBLK, block, None)


@functools.partial(
    pl.kernel,
    out_type=jax.ShapeDtypeStruct((E_PAD,), jnp.float32),
    mesh=_mesh,
    scratch_types=[
        pltpu.VMEM((EB,), jnp.int32),
        pltpu.VMEM((EB,), jnp.int32),
        pltpu.VMEM((EB,), jnp.float32),
        pltpu.VMEM((EB,), jnp.float32),
        pltpu.VMEM((EB,), jnp.float32),
        pltpu.VMEM((EB,), jnp.float32),
        pltpu.VMEM((EB,), jnp.float32),
        pltpu.VMEM((EB,), jnp.float32),
        pltpu.VMEM((EB,), jnp.float32),
        pltpu.SemaphoreType.DMA,
    ],
    compiler_params=_sc_params,
)
def _prep(px_hbm, py_hbm, pz_hbm, ii_hbm, ij_hbm, u_hbm,
          ii_v, ij_v, pix, piy, piz, pjx, pjy, pjz, u_v, sem):
    _prep_body(px_hbm, py_hbm, pz_hbm, ii_hbm, ij_hbm, u_hbm,
               ii_v, ij_v, pix, piy, piz, pjx, pjy, pjz, u_v, sem)


def _edge_body(u_hbm, ij_hbm, ii_hbm, xf_hbms, t_hbm, z_hbm, agg_hbm,
               t_v, ij_v, ii_v, u_v, ti_v, fr_v, xfj_v, msg_v, agg_shs,
               semg, sems):
    cid = lax.axis_index("c")
    sid = lax.axis_index("s")
    w = sid * NC + cid

    pltpu.sync_copy(t_hbm, t_v)
    for f in range(H):
        pltpu.sync_copy(z_hbm, agg_shs[f].at[pl.ds(sid * SPT, SPT)])
    plsc.subcore_barrier()

    def block(blk, _):
        ebase = w * EPW + blk * EB
        pltpu.sync_copy(ij_hbm.at[pl.ds(ebase, EB)], ij_v)
        pltpu.sync_copy(ii_hbm.at[pl.ds(ebase, EB)], ii_v)
        pltpu.sync_copy(u_hbm.at[pl.ds(ebase, EB)], u_v)

        def pre(g, _):
            sl = pl.ds(g * L, L)
            uv = u_v[sl]
            ki = jnp.minimum(uv.astype(jnp.int32), K - 1)
            fr_v[sl] = uv - ki.astype(jnp.float32)
            ti_v[sl] = ki * HP
            return _

        lax.fori_loop(0, NG, pre, None)

        gat = [None, None]
        sca = [None, None]
        gat[0] = pltpu.async_copy(xf_hbms[0].at[ij_v], xfj_v[0], semg[0])
        for p in range(H // 2):
            b = p % 2
            gat[b].wait()
            if p + 1 < H // 2:
                gat[1 - b] = pltpu.async_copy(
                    xf_hbms[p + 1].at[ij_v], xfj_v[1 - b], semg[1 - b])
            for d in sca:
                if d is not None:
                    d.wait()

            def group(g, _):
                sl = pl.ds(g * L, L)
                tb = ti_v[sl] + 2 * p
                fr = fr_v[sl]
                xi = plsc.bitcast(xfj_v[b][sl], jnp.int32)
                xe = plsc.bitcast(xi << 16, jnp.float32)
                xo = plsc.bitcast(xi & jnp.int32(-65536), jnp.float32)
                a0 = plsc.load_gather(t_v, [tb])
                b0 = plsc.load_gather(t_v, [tb + HP])
                a1 = plsc.load_gather(t_v, [tb + 1])
                b1 = plsc.load_gather(t_v, [tb + HP + 1])
                msg_v[0][sl] = xe * (a0 + fr * (b0 - a0))
                msg_v[1][sl] = xo * (a1 + fr * (b1 - a1))
                return _

            lax.fori_loop(0, NG, group, None)
            sca[0] = pltpu.async_copy(
                msg_v[0], agg_shs[2 * p].at[ii_v], sems[0], add=True)
            sca[1] = pltpu.async_copy(
                msg_v[1], agg_shs[2 * p + 1].at[ii_v], sems[1], add=True)
        for d in sca:
            if d is not None:
                d.wait()
        return _

    lax.fori_loop(0, NBLK, block, None)
    plsc.subcore_barrier()
    for f in range(H):
        pltpu.sync_copy(
            agg_shs[f].at[pl.ds(sid * SPT, SPT)],
            agg_hbm.at[pl.ds((cid * H + f) * NODE_PAD + sid * SPT, SPT)])


@functools.partial(
    pl.kernel,
    out_type=jax.ShapeDtypeStruct((NC * H * NODE_PAD,), jnp.float32),
    mesh=_mesh,
    scratch_types=[
        pltpu.VMEM((TROWS * HP,), jnp.float32),
        pltpu.VMEM((EB,), jnp.int32),
        pltpu.VMEM((EB,), jnp.int32),
        pltpu.VMEM((EB,), jnp.float32),
        pltpu.VMEM((EB,), jnp.int32),
        pltpu.VMEM((EB,), jnp.float32),
        pltpu.VMEM((EB,), jnp.float32),
        pltpu.VMEM((EB,), jnp.float32),
        pltpu.VMEM((EB,), jnp.float32),
        pltpu.VMEM((EB,), jnp.float32),
    ] + [pltpu.VMEM_SHARED((NODE_PAD,), jnp.float32)] * H + [
        pltpu.SemaphoreType.DMA,
        pltpu.SemaphoreType.DMA,
        pltpu.SemaphoreType.DMA,
        pltpu.SemaphoreType.DMA,
    ],
    compiler_params=_sc_params,
)
def _edge_pass(u_hbm, ij_hbm, ii_hbm, xf0, xf1, xf2, xf3, xf4,
               t_hbm, z_hbm, agg_hbm,
               t_v, ij_v, ii_v, u_v, ti_v, fr_v, xfj0, xfj1, msg0, msg1,
               a0, a1, a2, a3, a4, a5, a6, a7, a8, a9,
               semg0, semg1, sems0, sems1):
    _edge_body(u_hbm, ij_hbm, ii_hbm,
               [xf0, xf1, xf2, xf3, xf4],
               t_hbm, z_hbm, agg_hbm,
               t_v, ij_v, ii_v, u_v, ti_v, fr_v, [xfj0, xfj1], [msg0, msg1],
               [a0, a1, a2, a3, a4, a5, a6, a7, a8, a9],
               [semg0, semg1], [sems0, sems1])


def _node_body(nxt_w_ref, w1_ref, b1_ref, w2_ref, b2_ref, agg_ref, x_ref,
               xn_ref, xf_ref):
    cols = [agg_ref[f, 0, 0] + agg_ref[H + f, 0, 0] for f in range(H)]
    agg10 = jnp.stack(cols, axis=1)
    agg = jnp.concatenate(
        [agg10, jnp.zeros((2048, HP - H), jnp.float32)], axis=1)
    pre = jnp.dot(agg, w1_ref[...], preferred_element_type=jnp.float32) + b1_ref[...]
    v = jnp.dot(_sp(pre), w2_ref[...], preferred_element_type=jnp.float32) + b2_ref[...]
    xn = x_ref[...] + v
    xn_ref[...] = xn
    xf = jnp.dot(xn, nxt_w_ref[...], preferred_element_type=jnp.float32)
    _pack_pairs(xf, xf_ref)


def _node_pass(agg4, x, w1p, b1p, w2p, b2p, nxt_w):
    wspec = pl.BlockSpec((HP, HP), lambda i: (0, 0))
    bspec = pl.BlockSpec((HP,), lambda i: (0,))
    nspec = pl.BlockSpec((2048, HP), lambda i: (i, 0))
    return pl.pallas_call(
        _node_body,
        grid=(NB,),
        in_specs=[wspec, wspec, bspec, wspec, bspec,
                  pl.BlockSpec((2 * H, 1, 1, 2048), lambda i: (0, i, 0, 0)),
                  nspec],
        out_specs=[nspec,
                   pl.BlockSpec((H // 2, 1, 1, 2048), lambda i: (0, i, 0, 0))],
        out_shape=[jax.ShapeDtypeStruct((NODE_PAD, HP), jnp.float32),
                   jax.ShapeDtypeStruct((H // 2, NB, 1, 2048), jnp.float32)],
    )(nxt_w, w1p, b1p, w2p, b2p, agg4, x)


def _final_body(x_ref, xp_ref):
    i = pl.program_id(0)
    xn = x_ref[...]
    rowid = i * 2048 + lax.broadcasted_iota(jnp.int32, (2048, HP), 0)
    xn = jnp.where(rowid < N, xn, 0.0)
    for f in range(H):
        xp_ref[f, 0, 0] = xn[:, f]


def _final_planes(x):
    return pl.pallas_call(
        _final_body,
        grid=(NB,),
        in_specs=[pl.BlockSpec((2048, HP), lambda i: (i, 0))],
        out_specs=pl.BlockSpec((H, 1, 1, 2048), lambda i: (0, i, 0, 0)),
        out_shape=jax.ShapeDtypeStruct((H, NB, 1, 2048), jnp.float32),
    )(x)


def _readout_body(xp_hbm, b_hbm, z_hbm, one_hbm, mol_hbm,
                  b_v, x_v, one_v, mol_shs, sem):
    cid = lax.axis_index("c")
    sid = lax.axis_index("s")
    w = sid * NC + cid
    nsb = NODE_PAD // EB
    cnt = (nsb - 1 - w) // NW + 1
    for f in range(H + 1):
        pltpu.sync_copy(z_hbm, mol_shs[f].at[pl.ds(sid * MPT, MPT)])
    pltpu.sync_copy(one_hbm, one_v)
    plsc.subcore_barrier()

    def block(q, _):
        sb = w + q * NW
        pltpu.sync_copy(b_hbm.at[pl.ds(sb * EB, EB)], b_v)
        for f in range(H):
            pltpu.sync_copy(xp_hbm.at[pl.ds(f * NODE_PAD + sb * EB, EB)], x_v)
            pltpu.sync_copy(x_v, mol_shs[f].at[b_v], add=True)
        pltpu.sync_copy(one_v, mol_shs[H].at[b_v], add=True)
        return _

    lax.fori_loop(0, cnt, block, None)
    plsc.subcore_barrier()
    for f in range(H + 1):
        pltpu.sync_copy(
            mol_shs[f].at[pl.ds(sid * MPT, MPT)],
            mol_hbm.at[pl.ds((cid * (H + 1) + f) * MOL_PAD + sid * MPT, MPT)])


@functools.partial(
    pl.kernel,
    out_type=jax.ShapeDtypeStruct((NC * (H + 1) * MOL_PAD,), jnp.float32),
    mesh=_mesh,
    scratch_types=[
        pltpu.VMEM((EB,), jnp.int32),
        pltpu.VMEM((EB,), jnp.float32),
        pltpu.VMEM((EB,), jnp.float32),
    ] + [pltpu.VMEM_SHARED((MOL_PAD,), jnp.float32)] * (H + 1) + [
        pltpu.SemaphoreType.DMA,
    ],
    compiler_params=_sc_params,
)
def _readout(xp_hbm, b_hbm, z_hbm, one_hbm, mol_hbm, b_v, x_v, one_v,
             m0, m1, m2, m3, m4, m5, m6, m7, m8, m9, m10, sem):
    _readout_body(xp_hbm, b_hbm, z_hbm, one_hbm, mol_hbm, b_v, x_v, one_v,
                  [m0, m1, m2, m3, m4, m5, m6, m7, m8, m9, m10], sem)


def _head_body(molp_ref, w_ref, b_ref, out_ref):
    cols = [molp_ref[f] + molp_ref[H + 1 + f] for f in range(H + 1)]
    counts = jnp.maximum(cols[H], 1.0)[:, None]
    h10 = jnp.stack(cols[:H], axis=1) / counts
    out_ref[...] = jnp.dot(h10, w_ref[...],
                           preferred_element_type=jnp.float32) + b_ref[...]


def _head(molp, clf_w, clf_b):
    return pl.pallas_call(
        _head_body,
        out_shape=jax.ShapeDtypeStruct((MOL_PAD, NLAB), jnp.float32),
    )(molp, clf_w, clf_b)


def kernel(z, pos, batch, idx_i, idx_j, embedding, in2f_W, filt_W1, filt_b1,
           filt_W2, filt_b2, f2out_W1, f2out_b1, f2out_W2, f2out_b2, clf_W, clf_b):
    f32 = jnp.float32
    z3d = jnp.pad(z.astype(jnp.int32), (0, NODE_PAD - N)).reshape(-1, 1, 2048)
    posf = pos.astype(f32)
    px = jnp.pad(posf[:, 0], (0, NODE_PAD - N))
    py = jnp.pad(posf[:, 1], (0, NODE_PAD - N))
    pz = jnp.pad(posf[:, 2], (0, NODE_PAD - N))
    ii1 = jnp.pad(idx_i.astype(jnp.int32), (0, E_PAD - E))
    ij1 = jnp.pad(idx_j.astype(jnp.int32), (0, E_PAD - E))
    b1 = jnp.pad(batch.astype(jnp.int32), (0, NODE_PAD - N),
                 constant_values=NMOL)
    embp = jnp.pad(embedding.astype(f32), ((0, 128 - MAXZ), (0, HP - H)))
    in2fp = jnp.pad(in2f_W.astype(f32), ((0, 0), (0, HP - H), (0, HP - H)))
    fw2p = jnp.pad(filt_W2.astype(f32), ((0, 0), (0, 0), (0, HP - H)))
    fb2p = jnp.pad(filt_b2.astype(f32), ((0, 0), (0, HP - H)))
    ow1p = jnp.pad(f2out_W1.astype(f32), ((0, 0), (0, HP - H), (0, HP - H)))
    ob1p = jnp.pad(f2out_b1.astype(f32), ((0, 0), (0, HP - H)))
    ow2p = jnp.pad(f2out_W2.astype(f32), ((0, 0), (0, HP - H), (0, HP - H)))
    ob2p = jnp.pad(f2out_b2.astype(f32), ((0, 0), (0, HP - H)))
    zplane = jnp.zeros((SPT,), f32)
    zmol = jnp.zeros((MPT,), f32)
    ones = jnp.ones((EB,), f32)

    tables = _build_tables(filt_W1.astype(f32), filt_b1.astype(f32), fw2p, fb2p)
    tables2 = tables.reshape(NINT, TROWS * HP)
    x, xf4 = _x0_xf0(z3d, embp, in2fp[0])
    u = _prep(px, py, pz, ii1, ij1)
    nxt = jnp.roll(in2fp, -1, axis=0)

    niter = (jnp.float32(NINT) + jnp.min(posf) * 0.0).astype(jnp.int32)

    def step(t, carry):
        xc, xf4c = carry
        xfp = xf4c.reshape(H // 2, NODE_PAD)
        agg = _edge_pass(u, ij1, ii1, *[xfp[p] for p in range(H // 2)],
                         tables2[t], zplane)
        agg4 = agg.reshape(2 * H, NB, 1, 2048)
        xc, xf4c = _node_pass(agg4, xc, ow1p[t], ob1p[t], ow2p[t], ob2p[t],
                              nxt[t])
        return (xc, xf4c)

    x, _ = lax.fori_loop(0, niter, step, (x, xf4))
    xp = _final_planes(x).reshape(H * NODE_PAD)

    molp = _readout(xp, b1, zmol, ones)
    logits = _head(molp.reshape(NC * (H + 1), MOL_PAD), clf_W.astype(f32),
                   clf_b.astype(f32))
    return logits[:NMOL]


# speedup vs baseline: 6.8768x; 1.0008x over previous
import functools

import jax
import jax.numpy as jnp
from jax import lax
from jax.experimental import pallas as pl
from jax.experimental.pallas import tpu as pltpu
from jax.experimental.pallas import tpu_sc as plsc

N = 100000
E = 3200000
H = 10
NRBF = 30
NMOL = 5000
NLAB = 11
CUTOFF = 5.0
MAXZ = 100
NINT = 3

HP = 16
NC, NS, L = 2, 16, 16
NW = NC * NS
NODE_PAD = 100352
NB = NODE_PAD // 2048
SPT = NODE_PAD // NS
E_PAD = 3276800
EPW = E_PAD // NW
EB = 1024
NBLK = EPW // EB
NG = EB // L
K = 2048
TROWS = 2056
SCALE = K / CUTOFF
MOL_PAD = 6144
MPT = MOL_PAD // NS

_mesh = plsc.VectorSubcoreMesh(core_axis_name="c", subcore_axis_name="s")
_sc_params = pltpu.CompilerParams(needs_layout_passes=False)


def _sp(x):
    return jax.nn.softplus(x) - jnp.log(2.0)


def _tables_body(w1_ref, b1_ref, w2_ref, b2_ref, out_ref):
    dgrid = lax.broadcasted_iota(jnp.int32, (TROWS, 1), 0).astype(jnp.float32) * (
        CUTOFF / K)
    offs = lax.broadcasted_iota(jnp.int32, (1, NRBF), 1).astype(jnp.float32) * (
        CUTOFF / (NRBF - 1))
    width = CUTOFF / (NRBF - 1)
    coeff = -0.5 / (width * width)
    fg = jnp.exp(coeff * (dgrid - offs) ** 2)
    rc = 0.5 * (jnp.cos(dgrid * (jnp.pi / CUTOFF)) + 1.0)
    rc = rc * (dgrid < CUTOFF).astype(jnp.float32)
    for t in range(NINT):
        pre = jnp.dot(fg, w1_ref[t], preferred_element_type=jnp.float32) + b1_ref[t]
        wf = jnp.dot(_sp(pre), w2_ref[t], preferred_element_type=jnp.float32) + b2_ref[t]
        out_ref[t] = wf * rc


def _build_tables(w1, b1, w2p, b2p):
    return pl.pallas_call(
        _tables_body,
        out_shape=jax.ShapeDtypeStruct((NINT, TROWS, HP), jnp.float32),
    )(w1, b1, w2p, b2p)


def _pack_pairs(xf, xf_ref):
    u16 = lax.bitcast_convert_type(
        xf.astype(jnp.bfloat16), jnp.uint16).astype(jnp.uint32)
    for p in range(H // 2):
        pk = u16[:, 2 * p] | (u16[:, 2 * p + 1] << 16)
        xf_ref[p, 0, 0] = lax.bitcast_convert_type(pk, jnp.float32)


def _x0_body(z_ref, emb_ref, w_ref, x_ref, xf_ref):
    zv = z_ref[0, 0]
    oh = (zv[:, None] == lax.broadcasted_iota(jnp.int32, (2048, 128), 1))
    x0 = jnp.dot(oh.astype(jnp.float32), emb_ref[...],
                 preferred_element_type=jnp.float32)
    xf = jnp.dot(x0, w_ref[...], preferred_element_type=jnp.float32)
    x_ref[...] = x0
    _pack_pairs(xf, xf_ref)


def _x0_xf0(z3d, embp, in2f0):
    return pl.pallas_call(
        _x0_body,
        grid=(NB,),
        in_specs=[
            pl.BlockSpec((1, 1, 2048), lambda i: (i, 0, 0)),
            pl.BlockSpec((128, HP), lambda i: (0, 0)),
            pl.BlockSpec((HP, HP), lambda i: (0, 0)),
        ],
        out_specs=[
            pl.BlockSpec((2048, HP), lambda i: (i, 0)),
            pl.BlockSpec((H // 2, 1, 1, 2048), lambda i: (0, i, 0, 0)),
        ],
        out_shape=[
            jax.ShapeDtypeStruct((NODE_PAD, HP), jnp.float32),
            jax.ShapeDtypeStruct((H // 2, NB, 1, 2048), jnp.float32),
        ],
    )(z3d, embp, in2f0)


def _prep_body(px_hbm, py_hbm, pz_hbm, ii_hbm, ij_hbm, u_hbm,
               ii_vs, ij_vs, pbufs, u_vs, sems):
    cid = lax.axis_index("c")
    sid = lax.axis_index("s")
    w = sid * NC + cid
    iota = lax.iota(jnp.int32, L)
    kf32 = jnp.float32(K)

    def fire(blk, b):
        ebase = w * EPW + blk * EB
        pltpu.sync_copy(ii_hbm.at[pl.ds(ebase, EB)], ii_vs[b])
        pltpu.sync_copy(ij_hbm.at[pl.ds(ebase, EB)], ij_vs[b])
        pix, piy, piz, pjx, pjy, pjz = pbufs[b]
        return [
            pltpu.async_copy(px_hbm.at[ii_vs[b]], pix, sems[b]),
            pltpu.async_copy(py_hbm.at[ii_vs[b]], piy, sems[b]),
            pltpu.async_copy(pz_hbm.at[ii_vs[b]], piz, sems[b]),
            pltpu.async_copy(px_hbm.at[ij_vs[b]], pjx, sems[b]),
            pltpu.async_copy(py_hbm.at[ij_vs[b]], pjy, sems[b]),
            pltpu.async_copy(pz_hbm.at[ij_vs[b]], pjz, sems[b]),
        ]

    def compute(blk, b):
        pix, piy, piz, pjx, pjy, pjz = pbufs[b]
        u_v = u_vs[b]

        def group(g, _):
            sl = pl.ds(g * L, L)
            dx = pjx[sl] - pix[sl]
            dy = pjy[sl] - piy[sl]
            dz = pjz[sl] - piz[sl]
            s = dx * dx + dy * dy + dz * dz + 1e-12
            bi = plsc.bitcast(s, jnp.int32)
            y = plsc.bitcast((bi >> 1) + 0x1FBD1DF5, jnp.float32)
            y = 0.5 * (y + s / y)
            y = 0.5 * (y + s / y)
            y = 0.5 * (y + s / y)
            u = jnp.minimum(y * SCALE, kf32)
            gid = w * EPW + blk * EB + g * L + iota
            u = jnp.where(gid < E, u, kf32)
            u_v[sl] = u
            return _

        lax.fori_loop(0, NG, group, None)
        pltpu.sync_copy(u_v, u_hbm.at[pl.ds(w * EPW + blk * EB, EB)])

    descs = {0: fire(0, 0)}

    def super_block(sb, _):
        for b in range(2):
            blk = sb * 2 + b
            for d in descs[b]:
                d.wait()
            if b == 0:
                descs[1] = fire(blk + 1, 1)
            else:
                descs[0] = fire(jnp.minimum(blk + 1, NBLK - 1), 0)
            compute(blk, b)
        return _

    lax.fori_loop(0, NBLK // 2, super_block, None)
    for d in descs[0]:
        d.wait()


@functools.partial(
    pl.kernel,
    out_type=jax.ShapeDtypeStruct((E_PAD,), jnp.float32),
    mesh=_mesh,
    scratch_types=(
        [pltpu.VMEM((EB,), jnp.int32)] * 4
        + [pltpu.VMEM((EB,), jnp.float32)] * 14
        + [pltpu.SemaphoreType.DMA] * 2
    ),
    compiler_params=_sc_params,
)
def _prep(px_hbm, py_hbm, pz_hbm, ii_hbm, ij_hbm, u_hbm,
          ii0, ii1, ij0, ij1,
          p0, p1, p2, p3, p4, p5, q0, q1, q2, q3, q4, q5, u0, u1,
          sem0, sem1):
    _prep_body(px_hbm, py_hbm, pz_hbm, ii_hbm, ij_hbm, u_hbm,
               [ii0, ii1], [ij0, ij1],
               [[p0, p1, p2, p3, p4, p5], [q0, q1, q2, q3, q4, q5]],
               [u0, u1], [sem0, sem1])


def _edge_body(u_hbm, ij_hbm, ii_hbm, xf_hbms, t_hbm, z_hbm, agg_hbm,
               t_v, ij_v, ii_v, u_v, ti_v, fr_v, xfj_v, msg_v, agg_shs,
               semg, sems):
    cid = lax.axis_index("c")
    sid = lax.axis_index("s")
    w = sid * NC + cid

    pltpu.sync_copy(t_hbm, t_v)
    for f in range(H):
        pltpu.sync_copy(z_hbm, agg_shs[f].at[pl.ds(sid * SPT, SPT)])
    plsc.subcore_barrier()

    def block(blk, _):
        ebase = w * EPW + blk * EB
        pltpu.sync_copy(ij_hbm.at[pl.ds(ebase, EB)], ij_v)
        pltpu.sync_copy(ii_hbm.at[pl.ds(ebase, EB)], ii_v)
        pltpu.sync_copy(u_hbm.at[pl.ds(ebase, EB)], u_v)

        def pre(g, _):
            sl = pl.ds(g * L, L)
            uv = u_v[sl]
            ki = jnp.minimum(uv.astype(jnp.int32), K - 1)
            fr_v[sl] = uv - ki.astype(jnp.float32)
            ti_v[sl] = ki * HP
            return _

        lax.fori_loop(0, NG, pre, None)

        gat = [None, None]
        sca = [None, None]
        gat[0] = pltpu.async_copy(xf_hbms[0].at[ij_v], xfj_v[0], semg[0])
        for p in range(H // 2):
            b = p % 2
            gat[b].wait()
            if p + 1 < H // 2:
                gat[1 - b] = pltpu.async_copy(
                    xf_hbms[p + 1].at[ij_v], xfj_v[1 - b], semg[1 - b])
            for d in sca:
                if d is not None:
                    d.wait()

            def group(g, _):
                sl = pl.ds(g * L, L)
                tb = ti_v[sl] + 2 * p
                fr = fr_v[sl]
                xi = plsc.bitcast(xfj_v[b][sl], jnp.int32)
                xe = plsc.bitcast(xi << 16, jnp.float32)
                xo = plsc.bitcast(xi & jnp.int32(-65536), jnp.float32)
                a0 = plsc.load_gather(t_v, [tb])
                b0 = plsc.load_gather(t_v, [tb + HP])
                a1 = plsc.load_gather(t_v, [tb + 1])
                b1 = plsc.load_gather(t_v, [tb + HP + 1])
                msg_v[0][sl] = xe * (a0 + fr * (b0 - a0))
                msg_v[1][sl] = xo * (a1 + fr * (b1 - a1))
                return _

            lax.fori_loop(0, NG, group, None)
            sca[0] = pltpu.async_copy(
                msg_v[0], agg_shs[2 * p].at[ii_v], sems[0], add=True)
            sca[1] = pltpu.async_copy(
                msg_v[1], agg_shs[2 * p + 1].at[ii_v], sems[1], add=True)
        for d in sca:
            if d is not None:
                d.wait()
        return _

    lax.fori_loop(0, NBLK, block, None)
    plsc.subcore_barrier()
    for f in range(H):
        pltpu.sync_copy(
            agg_shs[f].at[pl.ds(sid * SPT, SPT)],
            agg_hbm.at[pl.ds((cid * H + f) * NODE_PAD + sid * SPT, SPT)])


@functools.partial(
    pl.kernel,
    out_type=jax.ShapeDtypeStruct((NC * H * NODE_PAD,), jnp.float32),
    mesh=_mesh,
    scratch_types=[
        pltpu.VMEM((TROWS * HP,), jnp.float32),
        pltpu.VMEM((EB,), jnp.int32),
        pltpu.VMEM((EB,), jnp.int32),
        pltpu.VMEM((EB,), jnp.float32),
        pltpu.VMEM((EB,), jnp.int32),
        pltpu.VMEM((EB,), jnp.float32),
        pltpu.VMEM((EB,), jnp.float32),
        pltpu.VMEM((EB,), jnp.float32),
        pltpu.VMEM((EB,), jnp.float32),
        pltpu.VMEM((EB,), jnp.float32),
    ] + [pltpu.VMEM_SHARED((NODE_PAD,), jnp.float32)] * H + [
        pltpu.SemaphoreType.DMA,
        pltpu.SemaphoreType.DMA,
        pltpu.SemaphoreType.DMA,
        pltpu.SemaphoreType.DMA,
    ],
    compiler_params=_sc_params,
)
def _edge_pass(u_hbm, ij_hbm, ii_hbm, xf0, xf1, xf2, xf3, xf4,
               t_hbm, z_hbm, agg_hbm,
               t_v, ij_v, ii_v, u_v, ti_v, fr_v, xfj0, xfj1, msg0, msg1,
               a0, a1, a2, a3, a4, a5, a6, a7, a8, a9,
               semg0, semg1, sems0, sems1):
    _edge_body(u_hbm, ij_hbm, ii_hbm,
               [xf0, xf1, xf2, xf3, xf4],
               t_hbm, z_hbm, agg_hbm,
               t_v, ij_v, ii_v, u_v, ti_v, fr_v, [xfj0, xfj1], [msg0, msg1],
               [a0, a1, a2, a3, a4, a5, a6, a7, a8, a9],
               [semg0, semg1], [sems0, sems1])


def _node_body(nxt_w_ref, w1_ref, b1_ref, w2_ref, b2_ref, agg_ref, x_ref,
               xn_ref, xf_ref):
    cols = [agg_ref[f, 0, 0] + agg_ref[H + f, 0, 0] for f in range(H)]
    agg10 = jnp.stack(cols, axis=1)
    agg = jnp.concatenate(
        [agg10, jnp.zeros((2048, HP - H), jnp.float32)], axis=1)
    pre = jnp.dot(agg, w1_ref[...], preferred_element_type=jnp.float32) + b1_ref[...]
    v = jnp.dot(_sp(pre), w2_ref[...], preferred_element_type=jnp.float32) + b2_ref[...]
    xn = x_ref[...] + v
    xn_ref[...] = xn
    xf = jnp.dot(xn, nxt_w_ref[...], preferred_element_type=jnp.float32)
    _pack_pairs(xf, xf_ref)


def _node_pass(agg4, x, w1p, b1p, w2p, b2p, nxt_w):
    wspec = pl.BlockSpec((HP, HP), lambda i: (0, 0))
    bspec = pl.BlockSpec((HP,), lambda i: (0,))
    nspec = pl.BlockSpec((2048, HP), lambda i: (i, 0))
    return pl.pallas_call(
        _node_body,
        grid=(NB,),
        in_specs=[wspec, wspec, bspec, wspec, bspec,
                  pl.BlockSpec((2 * H, 1, 1, 2048), lambda i: (0, i, 0, 0)),
                  nspec],
        out_specs=[nspec,
                   pl.BlockSpec((H // 2, 1, 1, 2048), lambda i: (0, i, 0, 0))],
        out_shape=[jax.ShapeDtypeStruct((NODE_PAD, HP), jnp.float32),
                   jax.ShapeDtypeStruct((H // 2, NB, 1, 2048), jnp.float32)],
    )(nxt_w, w1p, b1p, w2p, b2p, agg4, x)


def _final_body(x_ref, xp_ref):
    i = pl.program_id(0)
    xn = x_ref[...]
    rowid = i * 2048 + lax.broadcasted_iota(jnp.int32, (2048, HP), 0)
    xn = jnp.where(rowid < N, xn, 0.0)
    for f in range(H):
        xp_ref[f, 0, 0] = xn[:, f]


def _final_planes(x):
    return pl.pallas_call(
        _final_body,
        grid=(NB,),
        in_specs=[pl.BlockSpec((2048, HP), lambda i: (i, 0))],
        out_specs=pl.BlockSpec((H, 1, 1, 2048), lambda i: (0, i, 0, 0)),
        out_shape=jax.ShapeDtypeStruct((H, NB, 1, 2048), jnp.float32),
    )(x)


def _readout_body(xp_hbm, b_hbm, z_hbm, one_hbm, mol_hbm,
                  b_v, x_v, one_v, mol_shs, sem):
    cid = lax.axis_index("c")
    sid = lax.axis_index("s")
    w = sid * NC + cid
    nsb = NODE_PAD // EB
    cnt = (nsb - 1 - w) // NW + 1
    for f in range(H + 1):
        pltpu.sync_copy(z_hbm, mol_shs[f].at[pl.ds(sid * MPT, MPT)])
    pltpu.sync_copy(one_hbm, one_v)
    plsc.subcore_barrier()

    def block(q, _):
        sb = w + q * NW
        pltpu.sync_copy(b_hbm.at[pl.ds(sb * EB, EB)], b_v)
        for f in range(H):
            pltpu.sync_copy(xp_hbm.at[pl.ds(f * NODE_PAD + sb * EB, EB)], x_v)
            pltpu.sync_copy(x_v, mol_shs[f].at[b_v], add=True)
        pltpu.sync_copy(one_v, mol_shs[H].at[b_v], add=True)
        return _

    lax.fori_loop(0, cnt, block, None)
    plsc.subcore_barrier()
    for f in range(H + 1):
        pltpu.sync_copy(
            mol_shs[f].at[pl.ds(sid * MPT, MPT)],
            mol_hbm.at[pl.ds((cid * (H + 1) + f) * MOL_PAD + sid * MPT, MPT)])


@functools.partial(
    pl.kernel,
    out_type=jax.ShapeDtypeStruct((NC * (H + 1) * MOL_PAD,), jnp.float32),
    mesh=_mesh,
    scratch_types=[
        pltpu.VMEM((EB,), jnp.int32),
        pltpu.VMEM((EB,), jnp.float32),
        pltpu.VMEM((EB,), jnp.float32),
    ] + [pltpu.VMEM_SHARED((MOL_PAD,), jnp.float32)] * (H + 1) + [
        pltpu.SemaphoreType.DMA,
    ],
    compiler_params=_sc_params,
)
def _readout(xp_hbm, b_hbm, z_hbm, one_hbm, mol_hbm, b_v, x_v, one_v,
             m0, m1, m2, m3, m4, m5, m6, m7, m8, m9, m10, sem):
    _readout_body(xp_hbm, b_hbm, z_hbm, one_hbm, mol_hbm, b_v, x_v, one_v,
                  [m0, m1, m2, m3, m4, m5, m6, m7, m8, m9, m10], sem)


def _head_body(molp_ref, w_ref, b_ref, out_ref):
    cols = [molp_ref[f] + molp_ref[H + 1 + f] for f in range(H + 1)]
    counts = jnp.maximum(cols[H], 1.0)[:, None]
    h10 = jnp.stack(cols[:H], axis=1) / counts
    out_ref[...] = jnp.dot(h10, w_ref[...],
                           preferred_element_type=jnp.float32) + b_ref[...]


def _head(molp, clf_w, clf_b):
    return pl.pallas_call(
        _head_body,
        out_shape=jax.ShapeDtypeStruct((MOL_PAD, NLAB), jnp.float32),
    )(molp, clf_w, clf_b)


def kernel(z, pos, batch, idx_i, idx_j, embedding, in2f_W, filt_W1, filt_b1,
           filt_W2, filt_b2, f2out_W1, f2out_b1, f2out_W2, f2out_b2, clf_W, clf_b):
    f32 = jnp.float32
    z3d = jnp.pad(z.astype(jnp.int32), (0, NODE_PAD - N)).reshape(-1, 1, 2048)
    posf = pos.astype(f32)
    px = jnp.pad(posf[:, 0], (0, NODE_PAD - N))
    py = jnp.pad(posf[:, 1], (0, NODE_PAD - N))
    pz = jnp.pad(posf[:, 2], (0, NODE_PAD - N))
    ii1 = jnp.pad(idx_i.astype(jnp.int32), (0, E_PAD - E))
    ij1 = jnp.pad(idx_j.astype(jnp.int32), (0, E_PAD - E))
    b1 = jnp.pad(batch.astype(jnp.int32), (0, NODE_PAD - N),
                 constant_values=NMOL)
    embp = jnp.pad(embedding.astype(f32), ((0, 128 - MAXZ), (0, HP - H)))
    in2fp = jnp.pad(in2f_W.astype(f32), ((0, 0), (0, HP - H), (0, HP - H)))
    fw2p = jnp.pad(filt_W2.astype(f32), ((0, 0), (0, 0), (0, HP - H)))
    fb2p = jnp.pad(filt_b2.astype(f32), ((0, 0), (0, HP - H)))
    ow1p = jnp.pad(f2out_W1.astype(f32), ((0, 0), (0, HP - H), (0, HP - H)))
    ob1p = jnp.pad(f2out_b1.astype(f32), ((0, 0), (0, HP - H)))
    ow2p = jnp.pad(f2out_W2.astype(f32), ((0, 0), (0, HP - H), (0, HP - H)))
    ob2p = jnp.pad(f2out_b2.astype(f32), ((0, 0), (0, HP - H)))
    zplane = jnp.zeros((SPT,), f32)
    zmol = jnp.zeros((MPT,), f32)
    ones = jnp.ones((EB,), f32)

    tables = _build_tables(filt_W1.astype(f32), filt_b1.astype(f32), fw2p, fb2p)
    tables2 = tables.reshape(NINT, TROWS * HP)
    x, xf4 = _x0_xf0(z3d, embp, in2fp[0])
    u = _prep(px, py, pz, ii1, ij1)
    nxt = jnp.roll(in2fp, -1, axis=0)

    niter = (jnp.float32(NINT) + jnp.min(posf) * 0.0).astype(jnp.int32)

    def step(t, carry):
        xc, xf4c = carry
        xfp = xf4c.reshape(H // 2, NODE_PAD)
        agg = _edge_pass(u, ij1, ii1, *[xfp[p] for p in range(H // 2)],
                         tables2[t], zplane)
        agg4 = agg.reshape(2 * H, NB, 1, 2048)
        xc, xf4c = _node_pass(agg4, xc, ow1p[t], ob1p[t], ow2p[t], ob2p[t],
                              nxt[t])
        return (xc, xf4c)

    x, _ = lax.fori_loop(0, niter, step, (x, xf4))
    xp = _final_planes(x).reshape(H * NODE_PAD)

    molp = _readout(xp, b1, zmol, ones)
    logits = _head(molp.reshape(NC * (H + 1), MOL_PAD), clf_W.astype(f32),
                   clf_b.astype(f32))
    return logits[:NMOL]


# speedup vs baseline: 6.9387x; 1.0090x over previous
import functools

import jax
import jax.numpy as jnp
from jax import lax
from jax.experimental import pallas as pl
from jax.experimental.pallas import tpu as pltpu
from jax.experimental.pallas import tpu_sc as plsc

N = 100000
E = 3200000
H = 10
NRBF = 30
NMOL = 5000
NLAB = 11
CUTOFF = 5.0
MAXZ = 100
NINT = 3

HP = 16
NC, NS, L = 2, 16, 16
NW = NC * NS
NODE_PAD = 100352
NB = NODE_PAD // 2048
SPT = NODE_PAD // NS
E_PAD = 3276800
EPW = E_PAD // NW
EB = 1024
NBLK = EPW // EB
NG = EB // L
K = 2048
TROWS = 2056
SCALE = K / CUTOFF
MOL_PAD = 6144
MPT = MOL_PAD // NS

_mesh = plsc.VectorSubcoreMesh(core_axis_name="c", subcore_axis_name="s")
_sc_params = pltpu.CompilerParams(needs_layout_passes=False)


def _sp(x):
    return jax.nn.softplus(x) - jnp.log(2.0)


def _tables_body(w1_ref, b1_ref, w2_ref, b2_ref, out_ref):
    dgrid = lax.broadcasted_iota(jnp.int32, (TROWS, 1), 0).astype(jnp.float32) * (
        CUTOFF / K)
    offs = lax.broadcasted_iota(jnp.int32, (1, NRBF), 1).astype(jnp.float32) * (
        CUTOFF / (NRBF - 1))
    width = CUTOFF / (NRBF - 1)
    coeff = -0.5 / (width * width)
    fg = jnp.exp(coeff * (dgrid - offs) ** 2)
    rc = 0.5 * (jnp.cos(dgrid * (jnp.pi / CUTOFF)) + 1.0)
    rc = rc * (dgrid < CUTOFF).astype(jnp.float32)
    for t in range(NINT):
        pre = jnp.dot(fg, w1_ref[t], preferred_element_type=jnp.float32) + b1_ref[t]
        wf = jnp.dot(_sp(pre), w2_ref[t], preferred_element_type=jnp.float32) + b2_ref[t]
        out_ref[t] = wf * rc


def _build_tables(w1, b1, w2p, b2p):
    return pl.pallas_call(
        _tables_body,
        out_shape=jax.ShapeDtypeStruct((NINT, TROWS, HP), jnp.float32),
    )(w1, b1, w2p, b2p)


def _pack_pairs(xf, xf_ref):
    u16 = lax.bitcast_convert_type(
        xf.astype(jnp.bfloat16), jnp.uint16).astype(jnp.uint32)
    for p in range(H // 2):
        pk = u16[:, 2 * p] | (u16[:, 2 * p + 1] << 16)
        xf_ref[p, 0, 0] = lax.bitcast_convert_type(pk, jnp.float32)


def _x0_body(z_ref, emb_ref, w_ref, x_ref, xf_ref):
    zv = z_ref[0, 0]
    oh = (zv[:, None] == lax.broadcasted_iota(jnp.int32, (2048, 128), 1))
    x0 = jnp.dot(oh.astype(jnp.float32), emb_ref[...],
                 preferred_element_type=jnp.float32)
    xf = jnp.dot(x0, w_ref[...], preferred_element_type=jnp.float32)
    x_ref[...] = x0
    _pack_pairs(xf, xf_ref)


def _x0_xf0(z3d, embp, in2f0):
    return pl.pallas_call(
        _x0_body,
        grid=(NB,),
        in_specs=[
            pl.BlockSpec((1, 1, 2048), lambda i: (i, 0, 0)),
            pl.BlockSpec((128, HP), lambda i: (0, 0)),
            pl.BlockSpec((HP, HP), lambda i: (0, 0)),
        ],
        out_specs=[
            pl.BlockSpec((2048, HP), lambda i: (i, 0)),
            pl.BlockSpec((H // 2, 1, 1, 2048), lambda i: (0, i, 0, 0)),
        ],
        out_shape=[
            jax.ShapeDtypeStruct((NODE_PAD, HP), jnp.float32),
            jax.ShapeDtypeStruct((H // 2, NB, 1, 2048), jnp.float32),
        ],
    )(z3d, embp, in2f0)


def _prep_body(px_hbm, py_hbm, pz_hbm, ii_hbm, ij_hbm, u_hbm,
               ii_vs, ij_vs, pbufs, u_vs, sems):
    cid = lax.axis_index("c")
    sid = lax.axis_index("s")
    w = sid * NC + cid
    iota = lax.iota(jnp.int32, L)
    kf32 = jnp.float32(K)

    def fire(blk, b):
        ebase = w * EPW + blk * EB
        pltpu.sync_copy(ii_hbm.at[pl.ds(ebase, EB)], ii_vs[b])
        pltpu.sync_copy(ij_hbm.at[pl.ds(ebase, EB)], ij_vs[b])
        pix, piy, piz, pjx, pjy, pjz = pbufs[b]
        return [
            pltpu.async_copy(px_hbm.at[ii_vs[b]], pix, sems[b]),
            pltpu.async_copy(py_hbm.at[ii_vs[b]], piy, sems[b]),
            pltpu.async_copy(pz_hbm.at[ii_vs[b]], piz, sems[b]),
            pltpu.async_copy(px_hbm.at[ij_vs[b]], pjx, sems[b]),
            pltpu.async_copy(py_hbm.at[ij_vs[b]], pjy, sems[b]),
            pltpu.async_copy(pz_hbm.at[ij_vs[b]], pjz, sems[b]),
        ]

    def compute(blk, b):
        pix, piy, piz, pjx, pjy, pjz = pbufs[b]
        u_v = u_vs[b]

        def group(g, _):
            sl = pl.ds(g * L, L)
            dx = pjx[sl] - pix[sl]
            dy = pjy[sl] - piy[sl]
            dz = pjz[sl] - piz[sl]
            s = dx * dx + dy * dy + dz * dz + 1e-12
            bi = plsc.bitcast(s, jnp.int32)
            y = plsc.bitcast((bi >> 1) + 0x1FBD1DF5, jnp.float32)
            y = 0.5 * (y + s / y)
            y = 0.5 * (y + s / y)
            y = 0.5 * (y + s / y)
            u = jnp.minimum(y * SCALE, kf32)
            gid = w * EPW + blk * EB + g * L + iota
            u = jnp.where(gid < E, u, kf32)
            u_v[sl] = u
            return _

        lax.fori_loop(0, NG, group, None)
        pltpu.sync_copy(u_v, u_hbm.at[pl.ds(w * EPW + blk * EB, EB)])

    descs = {0: fire(0, 0)}

    def super_block(sb, _):
        for b in range(2):
            blk = sb * 2 + b
            for d in descs[b]:
                d.wait()
            if b == 0:
                descs[1] = fire(blk + 1, 1)
            else:
                descs[0] = fire(jnp.minimum(blk + 1, NBLK - 1), 0)
            compute(blk, b)
        return _

    lax.fori_loop(0, NBLK // 2, super_block, None)
    for d in descs[0]:
        d.wait()


@functools.partial(
    pl.kernel,
    out_type=jax.ShapeDtypeStruct((E_PAD,), jnp.float32),
    mesh=_mesh,
    scratch_types=(
        [pltpu.VMEM((EB,), jnp.int32)] * 4
        + [pltpu.VMEM((EB,), jnp.float32)] * 14
        + [pltpu.SemaphoreType.DMA] * 2
    ),
    compiler_params=_sc_params,
)
def _prep(px_hbm, py_hbm, pz_hbm, ii_hbm, ij_hbm, u_hbm,
          ii0, ii1, ij0, ij1,
          p0, p1, p2, p3, p4, p5, q0, q1, q2, q3, q4, q5, u0, u1,
          sem0, sem1):
    _prep_body(px_hbm, py_hbm, pz_hbm, ii_hbm, ij_hbm, u_hbm,
               [ii0, ii1], [ij0, ij1],
               [[p0, p1, p2, p3, p4, p5], [q0, q1, q2, q3, q4, q5]],
               [u0, u1], [sem0, sem1])


def _edge_body(u_hbm, ij_hbm, ii_hbm, xf_hbms, t_hbm, z_hbm, agg_hbm,
               t_v, ij_v, ii_v, u_v, ti_v, fr_v, xfj_v, msg_v, agg_shs,
               semg, sems):
    cid = lax.axis_index("c")
    sid = lax.axis_index("s")
    w = sid * NC + cid

    pltpu.sync_copy(t_hbm, t_v)
    for f in range(H):
        pltpu.sync_copy(z_hbm, agg_shs[f].at[pl.ds(sid * SPT, SPT)])
    plsc.subcore_barrier()

    def block(blk, _):
        ebase = w * EPW + blk * EB
        pltpu.sync_copy(ij_hbm.at[pl.ds(ebase, EB)], ij_v)
        pltpu.sync_copy(ii_hbm.at[pl.ds(ebase, EB)], ii_v)
        pltpu.sync_copy(u_hbm.at[pl.ds(ebase, EB)], u_v)

        def pre(g, _):
            sl = pl.ds(g * L, L)
            uv = u_v[sl]
            ki = jnp.minimum(uv.astype(jnp.int32), K - 1)
            fr_v[sl] = uv - ki.astype(jnp.float32)
            ti_v[sl] = ki * HP
            return _

        lax.fori_loop(0, NG, pre, None)

        gat = [None, None]
        sca = [None, None, None, None]
        gat[0] = pltpu.async_copy(xf_hbms[0].at[ij_v], xfj_v[0], semg[0])
        for p in range(H // 2):
            b = p % 2
            gat[b].wait()
            if p + 1 < H // 2:
                gat[1 - b] = pltpu.async_copy(
                    xf_hbms[p + 1].at[ij_v], xfj_v[1 - b], semg[1 - b])
            m0, m1 = 2 * (p % 2), 2 * (p % 2) + 1
            for mb in (m0, m1):
                if sca[mb] is not None:
                    sca[mb].wait()

            def group(g, _):
                sl = pl.ds(g * L, L)
                tb = ti_v[sl] + 2 * p
                fr = fr_v[sl]
                xi = plsc.bitcast(xfj_v[b][sl], jnp.int32)
                xe = plsc.bitcast(xi << 16, jnp.float32)
                xo = plsc.bitcast(xi & jnp.int32(-65536), jnp.float32)
                a0 = plsc.load_gather(t_v, [tb])
                b0 = plsc.load_gather(t_v, [tb + HP])
                a1 = plsc.load_gather(t_v, [tb + 1])
                b1 = plsc.load_gather(t_v, [tb + HP + 1])
                msg_v[m0][sl] = xe * (a0 + fr * (b0 - a0))
                msg_v[m1][sl] = xo * (a1 + fr * (b1 - a1))
                return _

            lax.fori_loop(0, NG, group, None)
            sca[m0] = pltpu.async_copy(
                msg_v[m0], agg_shs[2 * p].at[ii_v], sems[m0], add=True)
            sca[m1] = pltpu.async_copy(
                msg_v[m1], agg_shs[2 * p + 1].at[ii_v], sems[m1], add=True)
        for d in sca:
            if d is not None:
                d.wait()
        return _

    lax.fori_loop(0, NBLK, block, None)
    plsc.subcore_barrier()
    for f in range(H):
        pltpu.sync_copy(
            agg_shs[f].at[pl.ds(sid * SPT, SPT)],
            agg_hbm.at[pl.ds((cid * H + f) * NODE_PAD + sid * SPT, SPT)])


@functools.partial(
    pl.kernel,
    out_type=jax.ShapeDtypeStruct((NC * H * NODE_PAD,), jnp.float32),
    mesh=_mesh,
    scratch_types=[
        pltpu.VMEM((TROWS * HP,), jnp.float32),
        pltpu.VMEM((EB,), jnp.int32),
        pltpu.VMEM((EB,), jnp.int32),
        pltpu.VMEM((EB,), jnp.float32),
        pltpu.VMEM((EB,), jnp.int32),
        pltpu.VMEM((EB,), jnp.float32),
        pltpu.VMEM((EB,), jnp.float32),
        pltpu.VMEM((EB,), jnp.float32),
        pltpu.VMEM((EB,), jnp.float32),
        pltpu.VMEM((EB,), jnp.float32),
        pltpu.VMEM((EB,), jnp.float32),
        pltpu.VMEM((EB,), jnp.float32),
    ] + [pltpu.VMEM_SHARED((NODE_PAD,), jnp.float32)] * H + [
        pltpu.SemaphoreType.DMA,
        pltpu.SemaphoreType.DMA,
        pltpu.SemaphoreType.DMA,
        pltpu.SemaphoreType.DMA,
        pltpu.SemaphoreType.DMA,
        pltpu.SemaphoreType.DMA,
    ],
    compiler_params=_sc_params,
)
def _edge_pass(u_hbm, ij_hbm, ii_hbm, xf0, xf1, xf2, xf3, xf4,
               t_hbm, z_hbm, agg_hbm,
               t_v, ij_v, ii_v, u_v, ti_v, fr_v, xfj0, xfj1,
               msg0, msg1, msg2, msg3,
               a0, a1, a2, a3, a4, a5, a6, a7, a8, a9,
               semg0, semg1, sems0, sems1, sems2, sems3):
    _edge_body(u_hbm, ij_hbm, ii_hbm,
               [xf0, xf1, xf2, xf3, xf4],
               t_hbm, z_hbm, agg_hbm,
               t_v, ij_v, ii_v, u_v, ti_v, fr_v, [xfj0, xfj1],
               [msg0, msg1, msg2, msg3],
               [a0, a1, a2, a3, a4, a5, a6, a7, a8, a9],
               [semg0, semg1], [sems0, sems1, sems2, sems3])


def _node_body(nxt_w_ref, w1_ref, b1_ref, w2_ref, b2_ref, agg_ref, x_ref,
               xn_ref, xf_ref):
    cols = [agg_ref[f, 0, 0] + agg_ref[H + f, 0, 0] for f in range(H)]
    agg10 = jnp.stack(cols, axis=1)
    agg = jnp.concatenate(
        [agg10, jnp.zeros((2048, HP - H), jnp.float32)], axis=1)
    pre = jnp.dot(agg, w1_ref[...], preferred_element_type=jnp.float32) + b1_ref[...]
    v = jnp.dot(_sp(pre), w2_ref[...], preferred_element_type=jnp.float32) + b2_ref[...]
    xn = x_ref[...] + v
    xn_ref[...] = xn
    xf = jnp.dot(xn, nxt_w_ref[...], preferred_element_type=jnp.float32)
    _pack_pairs(xf, xf_ref)


def _node_pass(agg4, x, w1p, b1p, w2p, b2p, nxt_w):
    wspec = pl.BlockSpec((HP, HP), lambda i: (0, 0))
    bspec = pl.BlockSpec((HP,), lambda i: (0,))
    nspec = pl.BlockSpec((2048, HP), lambda i: (i, 0))
    return pl.pallas_call(
        _node_body,
        grid=(NB,),
        in_specs=[wspec, wspec, bspec, wspec, bspec,
                  pl.BlockSpec((2 * H, 1, 1, 2048), lambda i: (0, i, 0, 0)),
                  nspec],
        out_specs=[nspec,
                   pl.BlockSpec((H // 2, 1, 1, 2048), lambda i: (0, i, 0, 0))],
        out_shape=[jax.ShapeDtypeStruct((NODE_PAD, HP), jnp.float32),
                   jax.ShapeDtypeStruct((H // 2, NB, 1, 2048), jnp.float32)],
    )(nxt_w, w1p, b1p, w2p, b2p, agg4, x)


def _final_body(x_ref, xp_ref):
    i = pl.program_id(0)
    xn = x_ref[...]
    rowid = i * 2048 + lax.broadcasted_iota(jnp.int32, (2048, HP), 0)
    xn = jnp.where(rowid < N, xn, 0.0)
    for f in range(H):
        xp_ref[f, 0, 0] = xn[:, f]


def _final_planes(x):
    return pl.pallas_call(
        _final_body,
        grid=(NB,),
        in_specs=[pl.BlockSpec((2048, HP), lambda i: (i, 0))],
        out_specs=pl.BlockSpec((H, 1, 1, 2048), lambda i: (0, i, 0, 0)),
        out_shape=jax.ShapeDtypeStruct((H, NB, 1, 2048), jnp.float32),
    )(x)


def _readout_body(xp_hbm, b_hbm, z_hbm, one_hbm, mol_hbm,
                  b_v, x_v, one_v, mol_shs, sem):
    cid = lax.axis_index("c")
    sid = lax.axis_index("s")
    w = sid * NC + cid
    nsb = NODE_PAD // EB
    cnt = (nsb - 1 - w) // NW + 1
    for f in range(H + 1):
        pltpu.sync_copy(z_hbm, mol_shs[f].at[pl.ds(sid * MPT, MPT)])
    pltpu.sync_copy(one_hbm, one_v)
    plsc.subcore_barrier()

    def block(q, _):
        sb = w + q * NW
        pltpu.sync_copy(b_hbm.at[pl.ds(sb * EB, EB)], b_v)
        for f in range(H):
            pltpu.sync_copy(xp_hbm.at[pl.ds(f * NODE_PAD + sb * EB, EB)], x_v)
            pltpu.sync_copy(x_v, mol_shs[f].at[b_v], add=True)
        pltpu.sync_copy(one_v, mol_shs[H].at[b_v], add=True)
        return _

    lax.fori_loop(0, cnt, block, None)
    plsc.subcore_barrier()
    for f in range(H + 1):
        pltpu.sync_copy(
            mol_shs[f].at[pl.ds(sid * MPT, MPT)],
            mol_hbm.at[pl.ds((cid * (H + 1) + f) * MOL_PAD + sid * MPT, MPT)])


@functools.partial(
    pl.kernel,
    out_type=jax.ShapeDtypeStruct((NC * (H + 1) * MOL_PAD,), jnp.float32),
    mesh=_mesh,
    scratch_types=[
        pltpu.VMEM((EB,), jnp.int32),
        pltpu.VMEM((EB,), jnp.float32),
        pltpu.VMEM((EB,), jnp.float32),
    ] + [pltpu.VMEM_SHARED((MOL_PAD,), jnp.float32)] * (H + 1) + [
        pltpu.SemaphoreType.DMA,
    ],
    compiler_params=_sc_params,
)
def _readout(xp_hbm, b_hbm, z_hbm, one_hbm, mol_hbm, b_v, x_v, one_v,
             m0, m1, m2, m3, m4, m5, m6, m7, m8, m9, m10, sem):
    _readout_body(xp_hbm, b_hbm, z_hbm, one_hbm, mol_hbm, b_v, x_v, one_v,
                  [m0, m1, m2, m3, m4, m5, m6, m7, m8, m9, m10], sem)


def _head_body(molp_ref, w_ref, b_ref, out_ref):
    cols = [molp_ref[f] + molp_ref[H + 1 + f] for f in range(H + 1)]
    counts = jnp.maximum(cols[H], 1.0)[:, None]
    h10 = jnp.stack(cols[:H], axis=1) / counts
    out_ref[...] = jnp.dot(h10, w_ref[...],
                           preferred_element_type=jnp.float32) + b_ref[...]


def _head(molp, clf_w, clf_b):
    return pl.pallas_call(
        _head_body,
        out_shape=jax.ShapeDtypeStruct((MOL_PAD, NLAB), jnp.float32),
    )(molp, clf_w, clf_b)


def kernel(z, pos, batch, idx_i, idx_j, embedding, in2f_W, filt_W1, filt_b1,
           filt_W2, filt_b2, f2out_W1, f2out_b1, f2out_W2, f2out_b2, clf_W, clf_b):
    f32 = jnp.float32
    z3d = jnp.pad(z.astype(jnp.int32), (0, NODE_PAD - N)).reshape(-1, 1, 2048)
    posf = pos.astype(f32)
    px = jnp.pad(posf[:, 0], (0, NODE_PAD - N))
    py = jnp.pad(posf[:, 1], (0, NODE_PAD - N))
    pz = jnp.pad(posf[:, 2], (0, NODE_PAD - N))
    ii1 = jnp.pad(idx_i.astype(jnp.int32), (0, E_PAD - E))
    ij1 = jnp.pad(idx_j.astype(jnp.int32), (0, E_PAD - E))
    b1 = jnp.pad(batch.astype(jnp.int32), (0, NODE_PAD - N),
                 constant_values=NMOL)
    embp = jnp.pad(embedding.astype(f32), ((0, 128 - MAXZ), (0, HP - H)))
    in2fp = jnp.pad(in2f_W.astype(f32), ((0, 0), (0, HP - H), (0, HP - H)))
    fw2p = jnp.pad(filt_W2.astype(f32), ((0, 0), (0, 0), (0, HP - H)))
    fb2p = jnp.pad(filt_b2.astype(f32), ((0, 0), (0, HP - H)))
    ow1p = jnp.pad(f2out_W1.astype(f32), ((0, 0), (0, HP - H), (0, HP - H)))
    ob1p = jnp.pad(f2out_b1.astype(f32), ((0, 0), (0, HP - H)))
    ow2p = jnp.pad(f2out_W2.astype(f32), ((0, 0), (0, HP - H), (0, HP - H)))
    ob2p = jnp.pad(f2out_b2.astype(f32), ((0, 0), (0, HP - H)))
    zplane = jnp.zeros((SPT,), f32)
    zmol = jnp.zeros((MPT,), f32)
    ones = jnp.ones((EB,), f32)

    tables = _build_tables(filt_W1.astype(f32), filt_b1.astype(f32), fw2p, fb2p)
    tables2 = tables.reshape(NINT, TROWS * HP)
    x, xf4 = _x0_xf0(z3d, embp, in2fp[0])
    u = _prep(px, py, pz, ii1, ij1)
    nxt = jnp.roll(in2fp, -1, axis=0)

    niter = (jnp.float32(NINT) + jnp.min(posf) * 0.0).astype(jnp.int32)

    def step(t, carry):
        xc, xf4c = carry
        xfp = xf4c.reshape(H // 2, NODE_PAD)
        agg = _edge_pass(u, ij1, ii1, *[xfp[p] for p in range(H // 2)],
                         tables2[t], zplane)
        agg4 = agg.reshape(2 * H, NB, 1, 2048)
        xc, xf4c = _node_pass(agg4, xc, ow1p[t], ob1p[t], ow2p[t], ob2p[t],
                              nxt[t])
        return (xc, xf4c)

    x, _ = lax.fori_loop(0, niter, step, (x, xf4))
    xp = _final_planes(x).reshape(H * NODE_PAD)

    molp = _readout(xp, b1, zmol, ones)
    logits = _head(molp.reshape(NC * (H + 1), MOL_PAD), clf_W.astype(f32),
                   clf_b.astype(f32))
    return logits[:NMOL]


# speedup vs baseline: 6.9605x; 1.0031x over previous
import functools

import jax
import jax.numpy as jnp
from jax import lax
from jax.experimental import pallas as pl
from jax.experimental.pallas import tpu as pltpu
from jax.experimental.pallas import tpu_sc as plsc

N = 100000
E = 3200000
H = 10
NRBF = 30
NMOL = 5000
NLAB = 11
CUTOFF = 5.0
MAXZ = 100
NINT = 3

HP = 16
NC, NS, L = 2, 16, 16
NW = NC * NS
NODE_PAD = 100352
NB = NODE_PAD // 2048
SPT = NODE_PAD // NS
E_PAD = 3276800
EPW = E_PAD // NW
EB = 1024
NBLK = EPW // EB
NG = EB // L
K = 2048
TROWS = 2056
SCALE = K / CUTOFF
MOL_PAD = 6144
MPT = MOL_PAD // NS

_mesh = plsc.VectorSubcoreMesh(core_axis_name="c", subcore_axis_name="s",
                               num_cores=NC, num_subcores=NS)
_sc_params = pltpu.CompilerParams(needs_layout_passes=False)


def _sp(x):
    return jax.nn.softplus(x) - jnp.log(2.0)


def _tables_body(w1_ref, b1_ref, w2_ref, b2_ref, out_ref):
    dgrid = lax.broadcasted_iota(jnp.int32, (TROWS, 1), 0).astype(jnp.float32) * (
        CUTOFF / K)
    offs = lax.broadcasted_iota(jnp.int32, (1, NRBF), 1).astype(jnp.float32) * (
        CUTOFF / (NRBF - 1))
    width = CUTOFF / (NRBF - 1)
    coeff = -0.5 / (width * width)
    fg = jnp.exp(coeff * (dgrid - offs) ** 2)
    rc = 0.5 * (jnp.cos(dgrid * (jnp.pi / CUTOFF)) + 1.0)
    rc = rc * (dgrid < CUTOFF).astype(jnp.float32)
    for t in range(NINT):
        pre = jnp.dot(fg, w1_ref[t], preferred_element_type=jnp.float32) + b1_ref[t]
        wf = jnp.dot(_sp(pre), w2_ref[t], preferred_element_type=jnp.float32) + b2_ref[t]
        out_ref[t] = wf * rc


def _build_tables(w1, b1, w2p, b2p):
    return pl.pallas_call(
        _tables_body,
        out_shape=jax.ShapeDtypeStruct((NINT, TROWS, HP), jnp.float32),
    )(w1, b1, w2p, b2p)


def _pack_pairs(xf, xf_ref):
    u16 = lax.bitcast_convert_type(
        xf.astype(jnp.bfloat16), jnp.uint16).astype(jnp.uint32)
    for p in range(H // 2):
        pk = u16[:, 2 * p] | (u16[:, 2 * p + 1] << 16)
        xf_ref[p, 0, 0] = lax.bitcast_convert_type(pk, jnp.float32)


def _x0_body(z_ref, emb_ref, w_ref, x_ref, xf_ref):
    zv = z_ref[0, 0]
    oh = (zv[:, None] == lax.broadcasted_iota(jnp.int32, (2048, 128), 1))
    x0 = jnp.dot(oh.astype(jnp.float32), emb_ref[...],
                 preferred_element_type=jnp.float32)
    xf = jnp.dot(x0, w_ref[...], preferred_element_type=jnp.float32)
    x_ref[...] = x0
    _pack_pairs(xf, xf_ref)


def _x0_xf0(z3d, embp, in2f0):
    return pl.pallas_call(
        _x0_body,
        grid=(NB,),
        in_specs=[
            pl.BlockSpec((1, 1, 2048), lambda i: (i, 0, 0)),
            pl.BlockSpec((128, HP), lambda i: (0, 0)),
            pl.BlockSpec((HP, HP), lambda i: (0, 0)),
        ],
        out_specs=[
            pl.BlockSpec((2048, HP), lambda i: (i, 0)),
            pl.BlockSpec((H // 2, 1, 1, 2048), lambda i: (0, i, 0, 0)),
        ],
        out_shape=[
            jax.ShapeDtypeStruct((NODE_PAD, HP), jnp.float32),
            jax.ShapeDtypeStruct((H // 2, NB, 1, 2048), jnp.float32),
        ],
    )(z3d, embp, in2f0)


def _prep_body(px_hbm, py_hbm, pz_hbm, ii_hbm, ij_hbm, u_hbm,
               ii_vs, ij_vs, pbufs, u_vs, sems):
    cid = lax.axis_index("c")
    sid = lax.axis_index("s")
    w = sid * NC + cid
    iota = lax.iota(jnp.int32, L)
    kf32 = jnp.float32(K)

    def fire(blk, b):
        ebase = w * EPW + blk * EB
        pltpu.sync_copy(ii_hbm.at[pl.ds(ebase, EB)], ii_vs[b])
        pltpu.sync_copy(ij_hbm.at[pl.ds(ebase, EB)], ij_vs[b])
        pix, piy, piz, pjx, pjy, pjz = pbufs[b]
        return [
            pltpu.async_copy(px_hbm.at[ii_vs[b]], pix, sems[b]),
            pltpu.async_copy(py_hbm.at[ii_vs[b]], piy, sems[b]),
            pltpu.async_copy(pz_hbm.at[ii_vs[b]], piz, sems[b]),
            pltpu.async_copy(px_hbm.at[ij_vs[b]], pjx, sems[b]),
            pltpu.async_copy(py_hbm.at[ij_vs[b]], pjy, sems[b]),
            pltpu.async_copy(pz_hbm.at[ij_vs[b]], pjz, sems[b]),
        ]

    def compute(blk, b):
        pix, piy, piz, pjx, pjy, pjz = pbufs[b]
        u_v = u_vs[b]

        def group(g, _):
            sl = pl.ds(g * L, L)
            dx = pjx[sl] - pix[sl]
            dy = pjy[sl] - piy[sl]
            dz = pjz[sl] - piz[sl]
            s = dx * dx + dy * dy + dz * dz + 1e-12
            bi = plsc.bitcast(s, jnp.int32)
            y = plsc.bitcast((bi >> 1) + 0x1FBD1DF5, jnp.float32)
            y = 0.5 * (y + s / y)
            y = 0.5 * (y + s / y)
            y = 0.5 * (y + s / y)
            u = jnp.minimum(y * SCALE, kf32)
            gid = w * EPW + blk * EB + g * L + iota
            u = jnp.where(gid < E, u, kf32)
            u_v[sl] = u
            return _

        lax.fori_loop(0, NG, group, None)
        pltpu.sync_copy(u_v, u_hbm.at[pl.ds(w * EPW + blk * EB, EB)])

    descs = {0: fire(0, 0)}

    def super_block(sb, _):
        for b in range(2):
            blk = sb * 2 + b
            for d in descs[b]:
                d.wait()
            if b == 0:
                descs[1] = fire(blk + 1, 1)
            else:
                descs[0] = fire(jnp.minimum(blk + 1, NBLK - 1), 0)
            compute(blk, b)
        return _

    lax.fori_loop(0, NBLK // 2, super_block, None)
    for d in descs[0]:
        d.wait()


@functools.partial(
    pl.kernel,
    out_type=jax.ShapeDtypeStruct((E_PAD,), jnp.float32),
    mesh=_mesh,
    scratch_types=(
        [pltpu.VMEM((EB,), jnp.int32)] * 4
        + [pltpu.VMEM((EB,), jnp.float32)] * 14
        + [pltpu.SemaphoreType.DMA] * 2
    ),
    compiler_params=_sc_params,
)
def _prep(px_hbm, py_hbm, pz_hbm, ii_hbm, ij_hbm, u_hbm,
          ii0, ii1, ij0, ij1,
          p0, p1, p2, p3, p4, p5, q0, q1, q2, q3, q4, q5, u0, u1,
          sem0, sem1):
    _prep_body(px_hbm, py_hbm, pz_hbm, ii_hbm, ij_hbm, u_hbm,
               [ii0, ii1], [ij0, ij1],
               [[p0, p1, p2, p3, p4, p5], [q0, q1, q2, q3, q4, q5]],
               [u0, u1], [sem0, sem1])


def _edge_body(u_hbm, ij_hbm, ii_hbm, xf_hbms, t_hbm, z_hbm, agg_hbm,
               t_v, ij_v, ii_v, u_v, ti_v, fr_v, xfj_v, msg_v, agg_shs,
               semg, sems):
    cid = lax.axis_index("c")
    sid = lax.axis_index("s")
    w = sid * NC + cid

    pltpu.sync_copy(t_hbm, t_v)
    for f in range(H):
        pltpu.sync_copy(z_hbm, agg_shs[f].at[pl.ds(sid * SPT, SPT)])
    plsc.subcore_barrier()

    def block(blk, _):
        ebase = w * EPW + blk * EB
        pltpu.sync_copy(ij_hbm.at[pl.ds(ebase, EB)], ij_v)
        pltpu.sync_copy(ii_hbm.at[pl.ds(ebase, EB)], ii_v)
        pltpu.sync_copy(u_hbm.at[pl.ds(ebase, EB)], u_v)

        def pre(g, _):
            sl = pl.ds(g * L, L)
            uv = u_v[sl]
            ki = jnp.minimum(uv.astype(jnp.int32), K - 1)
            fr_v[sl] = uv - ki.astype(jnp.float32)
            ti_v[sl] = ki * HP
            return _

        lax.fori_loop(0, NG, pre, None)

        gat = [None, None]
        sca = [None, None, None, None]
        gat[0] = pltpu.async_copy(xf_hbms[0].at[ij_v], xfj_v[0], semg[0])
        for p in range(H // 2):
            b = p % 2
            gat[b].wait()
            if p + 1 < H // 2:
                gat[1 - b] = pltpu.async_copy(
                    xf_hbms[p + 1].at[ij_v], xfj_v[1 - b], semg[1 - b])
            m0, m1 = 2 * (p % 2), 2 * (p % 2) + 1
            for mb in (m0, m1):
                if sca[mb] is not None:
                    sca[mb].wait()

            def group(g, _):
                sl = pl.ds(g * L, L)
                tb = ti_v[sl] + 2 * p
                fr = fr_v[sl]
                xi = plsc.bitcast(xfj_v[b][sl], jnp.int32)
                xe = plsc.bitcast(xi << 16, jnp.float32)
                xo = plsc.bitcast(xi & jnp.int32(-65536), jnp.float32)
                a0 = plsc.load_gather(t_v, [tb])
                b0 = plsc.load_gather(t_v, [tb + HP])
                a1 = plsc.load_gather(t_v, [tb + 1])
                b1 = plsc.load_gather(t_v, [tb + HP + 1])
                msg_v[m0][sl] = xe * (a0 + fr * (b0 - a0))
                msg_v[m1][sl] = xo * (a1 + fr * (b1 - a1))
                return _

            lax.fori_loop(0, NG, group, None)
            sca[m0] = pltpu.async_copy(
                msg_v[m0], agg_shs[2 * p].at[ii_v], sems[m0], add=True)
            sca[m1] = pltpu.async_copy(
                msg_v[m1], agg_shs[2 * p + 1].at[ii_v], sems[m1], add=True)
        for d in sca:
            if d is not None:
                d.wait()
        return _

    lax.fori_loop(0, NBLK, block, None)
    plsc.subcore_barrier()
    for f in range(H):
        pltpu.sync_copy(
            agg_shs[f].at[pl.ds(sid * SPT, SPT)],
            agg_hbm.at[pl.ds((cid * H + f) * NODE_PAD + sid * SPT, SPT)])


@functools.partial(
    pl.kernel,
    out_type=jax.ShapeDtypeStruct((NC * H * NODE_PAD,), jnp.float32),
    mesh=_mesh,
    scratch_types=[
        pltpu.VMEM((TROWS * HP,), jnp.float32),
        pltpu.VMEM((EB,), jnp.int32),
        pltpu.VMEM((EB,), jnp.int32),
        pltpu.VMEM((EB,), jnp.float32),
        pltpu.VMEM((EB,), jnp.int32),
        pltpu.VMEM((EB,), jnp.float32),
        pltpu.VMEM((EB,), jnp.float32),
        pltpu.VMEM((EB,), jnp.float32),
        pltpu.VMEM((EB,), jnp.float32),
        pltpu.VMEM((EB,), jnp.float32),
        pltpu.VMEM((EB,), jnp.float32),
        pltpu.VMEM((EB,), jnp.float32),
    ] + [pltpu.VMEM_SHARED((NODE_PAD,), jnp.float32)] * H + [
        pltpu.SemaphoreType.DMA,
        pltpu.SemaphoreType.DMA,
        pltpu.SemaphoreType.DMA,
        pltpu.SemaphoreType.DMA,
        pltpu.SemaphoreType.DMA,
        pltpu.SemaphoreType.DMA,
    ],
    compiler_params=_sc_params,
)
def _edge_pass(u_hbm, ij_hbm, ii_hbm, xf0, xf1, xf2, xf3, xf4,
               t_hbm, z_hbm, agg_hbm,
               t_v, ij_v, ii_v, u_v, ti_v, fr_v, xfj0, xfj1,
               msg0, msg1, msg2, msg3,
               a0, a1, a2, a3, a4, a5, a6, a7, a8, a9,
               semg0, semg1, sems0, sems1, sems2, sems3):
    _edge_body(u_hbm, ij_hbm, ii_hbm,
               [xf0, xf1, xf2, xf3, xf4],
               t_hbm, z_hbm, agg_hbm,
               t_v, ij_v, ii_v, u_v, ti_v, fr_v, [xfj0, xfj1],
               [msg0, msg1, msg2, msg3],
               [a0, a1, a2, a3, a4, a5, a6, a7, a8, a9],
               [semg0, semg1], [sems0, sems1, sems2, sems3])


def _node_body(nxt_w_ref, w1_ref, b1_ref, w2_ref, b2_ref, agg_ref, x_ref,
               xn_ref, xf_ref):
    cols = [agg_ref[f, 0, 0] + agg_ref[H + f, 0, 0] for f in range(H)]
    agg10 = jnp.stack(cols, axis=1)
    agg = jnp.concatenate(
        [agg10, jnp.zeros((2048, HP - H), jnp.float32)], axis=1)
    pre = jnp.dot(agg, w1_ref[...], preferred_element_type=jnp.float32) + b1_ref[...]
    v = jnp.dot(_sp(pre), w2_ref[...], preferred_element_type=jnp.float32) + b2_ref[...]
    xn = x_ref[...] + v
    xn_ref[...] = xn
    xf = jnp.dot(xn, nxt_w_ref[...], preferred_element_type=jnp.float32)
    _pack_pairs(xf, xf_ref)


def _node_pass(agg4, x, w1p, b1p, w2p, b2p, nxt_w):
    wspec = pl.BlockSpec((HP, HP), lambda i: (0, 0))
    bspec = pl.BlockSpec((HP,), lambda i: (0,))
    nspec = pl.BlockSpec((2048, HP), lambda i: (i, 0))
    return pl.pallas_call(
        _node_body,
        grid=(NB,),
        in_specs=[wspec, wspec, bspec, wspec, bspec,
                  pl.BlockSpec((2 * H, 1, 1, 2048), lambda i: (0, i, 0, 0)),
                  nspec],
        out_specs=[nspec,
                   pl.BlockSpec((H // 2, 1, 1, 2048), lambda i: (0, i, 0, 0))],
        out_shape=[jax.ShapeDtypeStruct((NODE_PAD, HP), jnp.float32),
                   jax.ShapeDtypeStruct((H // 2, NB, 1, 2048), jnp.float32)],
    )(nxt_w, w1p, b1p, w2p, b2p, agg4, x)


def _final_body(x_ref, xp_ref):
    i = pl.program_id(0)
    xn = x_ref[...]
    rowid = i * 2048 + lax.broadcasted_iota(jnp.int32, (2048, HP), 0)
    xn = jnp.where(rowid < N, xn, 0.0)
    for f in range(H):
        xp_ref[f, 0, 0] = xn[:, f]


def _final_planes(x):
    return pl.pallas_call(
        _final_body,
        grid=(NB,),
        in_specs=[pl.BlockSpec((2048, HP), lambda i: (i, 0))],
        out_specs=pl.BlockSpec((H, 1, 1, 2048), lambda i: (0, i, 0, 0)),
        out_shape=jax.ShapeDtypeStruct((H, NB, 1, 2048), jnp.float32),
    )(x)


def _readout_body(xp_hbm, b_hbm, z_hbm, one_hbm, mol_hbm,
                  b_v, x_v, one_v, mol_shs, sem):
    cid = lax.axis_index("c")
    sid = lax.axis_index("s")
    w = sid * NC + cid
    nsb = NODE_PAD // EB
    cnt = (nsb - 1 - w) // NW + 1
    for f in range(H + 1):
        pltpu.sync_copy(z_hbm, mol_shs[f].at[pl.ds(sid * MPT, MPT)])
    pltpu.sync_copy(one_hbm, one_v)
    plsc.subcore_barrier()

    def block(q, _):
        sb = w + q * NW
        pltpu.sync_copy(b_hbm.at[pl.ds(sb * EB, EB)], b_v)
        for f in range(H):
            pltpu.sync_copy(xp_hbm.at[pl.ds(f * NODE_PAD + sb * EB, EB)], x_v)
            pltpu.sync_copy(x_v, mol_shs[f].at[b_v], add=True)
        pltpu.sync_copy(one_v, mol_shs[H].at[b_v], add=True)
        return _

    lax.fori_loop(0, cnt, block, None)
    plsc.subcore_barrier()
    for f in range(H + 1):
        pltpu.sync_copy(
            mol_shs[f].at[pl.ds(sid * MPT, MPT)],
            mol_hbm.at[pl.ds((cid * (H + 1) + f) * MOL_PAD + sid * MPT, MPT)])


@functools.partial(
    pl.kernel,
    out_type=jax.ShapeDtypeStruct((NC * (H + 1) * MOL_PAD,), jnp.float32),
    mesh=_mesh,
    scratch_types=[
        pltpu.VMEM((EB,), jnp.int32),
        pltpu.VMEM((EB,), jnp.float32),
        pltpu.VMEM((EB,), jnp.float32),
    ] + [pltpu.VMEM_SHARED((MOL_PAD,), jnp.float32)] * (H + 1) + [
        pltpu.SemaphoreType.DMA,
    ],
    compiler_params=_sc_params,
)
def _readout(xp_hbm, b_hbm, z_hbm, one_hbm, mol_hbm, b_v, x_v, one_v,
             m0, m1, m2, m3, m4, m5, m6, m7, m8, m9, m10, sem):
    _readout_body(xp_hbm, b_hbm, z_hbm, one_hbm, mol_hbm, b_v, x_v, one_v,
                  [m0, m1, m2, m3, m4, m5, m6, m7, m8, m9, m10], sem)


def _head_body(molp_ref, w_ref, b_ref, out_ref):
    cols = [molp_ref[f] + molp_ref[H + 1 + f] for f in range(H + 1)]
    counts = jnp.maximum(cols[H], 1.0)[:, None]
    h10 = jnp.stack(cols[:H], axis=1) / counts
    out_ref[...] = jnp.dot(h10, w_ref[...],
                           preferred_element_type=jnp.float32) + b_ref[...]


def _head(molp, clf_w, clf_b):
    return pl.pallas_call(
        _head_body,
        out_shape=jax.ShapeDtypeStruct((MOL_PAD, NLAB), jnp.float32),
    )(molp, clf_w, clf_b)


def kernel(z, pos, batch, idx_i, idx_j, embedding, in2f_W, filt_W1, filt_b1,
           filt_W2, filt_b2, f2out_W1, f2out_b1, f2out_W2, f2out_b2, clf_W, clf_b):
    f32 = jnp.float32
    z3d = jnp.pad(z.astype(jnp.int32), (0, NODE_PAD - N)).reshape(-1, 1, 2048)
    posf = pos.astype(f32)
    px = jnp.pad(posf[:, 0], (0, NODE_PAD - N))
    py = jnp.pad(posf[:, 1], (0, NODE_PAD - N))
    pz = jnp.pad(posf[:, 2], (0, NODE_PAD - N))
    ii1 = jnp.pad(idx_i.astype(jnp.int32), (0, E_PAD - E))
    ij1 = jnp.pad(idx_j.astype(jnp.int32), (0, E_PAD - E))
    b1 = jnp.pad(batch.astype(jnp.int32), (0, NODE_PAD - N),
                 constant_values=NMOL)
    embp = jnp.pad(embedding.astype(f32), ((0, 128 - MAXZ), (0, HP - H)))
    in2fp = jnp.pad(in2f_W.astype(f32), ((0, 0), (0, HP - H), (0, HP - H)))
    fw2p = jnp.pad(filt_W2.astype(f32), ((0, 0), (0, 0), (0, HP - H)))
    fb2p = jnp.pad(filt_b2.astype(f32), ((0, 0), (0, HP - H)))
    ow1p = jnp.pad(f2out_W1.astype(f32), ((0, 0), (0, HP - H), (0, HP - H)))
    ob1p = jnp.pad(f2out_b1.astype(f32), ((0, 0), (0, HP - H)))
    ow2p = jnp.pad(f2out_W2.astype(f32), ((0, 0), (0, HP - H), (0, HP - H)))
    ob2p = jnp.pad(f2out_b2.astype(f32), ((0, 0), (0, HP - H)))
    zplane = jnp.zeros((SPT,), f32)
    zmol = jnp.zeros((MPT,), f32)
    ones = jnp.ones((EB,), f32)

    tables = _build_tables(filt_W1.astype(f32), filt_b1.astype(f32), fw2p, fb2p)
    tables2 = tables.reshape(NINT, TROWS * HP)
    x, xf4 = _x0_xf0(z3d, embp, in2fp[0])
    u = _prep(px, py, pz, ii1, ij1)
    nxt = jnp.roll(in2fp, -1, axis=0)

    niter = (jnp.float32(NINT) + jnp.min(posf) * 0.0).astype(jnp.int32)

    def step(t, carry):
        xc, xf4c = carry
        xfp = xf4c.reshape(H // 2, NODE_PAD)
        agg = _edge_pass(u, ij1, ii1, *[xfp[p] for p in range(H // 2)],
                         tables2[t], zplane)
        agg4 = agg.reshape(2 * H, NB, 1, 2048)
        xc, xf4c = _node_pass(agg4, xc, ow1p[t], ob1p[t], ow2p[t], ob2p[t],
                              nxt[t])
        return (xc, xf4c)

    x, _ = lax.fori_loop(0, niter, step, (x, xf4))
    xp = _final_planes(x).reshape(H * NODE_PAD)

    molp = _readout(xp, b1, zmol, ones)
    logits = _head(molp.reshape(NC * (H + 1), MOL_PAD), clf_W.astype(f32),
                   clf_b.astype(f32))
    return logits[:NMOL]
